# Initial kernel scaffold; baseline (speedup 1.0000x reference)
import jax
import jax.numpy as jnp
from jax.experimental import pallas as pl


def kernel(x, edge_index, edge_type, W1, root1, b1, W2, root2, b2):
    raise NotImplementedError("write your pallas kernel here")


# speedup vs baseline: 12.2826x; 12.2826x over previous
import functools

import jax
import jax.numpy as jnp
from jax import lax
from jax.experimental import pallas as pl
from jax.experimental.pallas import tpu as pltpu
from jax.experimental.pallas import tpu_sc as plsc

N_NODES = 10000
N_EDGES = 320000
D_IN = 128
D_HID = 128
D_OUT = 64
N_REL = 8

NC, NS = 2, 16
NW = NC * NS
E_TILE = N_EDGES // NW
E_CNT = N_EDGES // NS
KEYS = N_NODES * N_REL
KEYS_PAD = 81920
RED = KEYS_PAD // NS
CHUNK = 80
BLK = 2000
ROWS_TILE = N_NODES // NS

_mesh = plsc.VectorSubcoreMesh(core_axis_name="c", subcore_axis_name="s")
_sc_params = pltpu.CompilerParams(needs_layout_passes=False)

_Z16 = functools.partial(jnp.zeros, (16,), jnp.float32)
_O16 = functools.partial(jnp.ones, (16,), jnp.float32)


def _scales_body(et_hbm, dst_hbm, s_hbm,
                 cnt_full, cnt_slice, tbuf, dbuf, sbuf, fin_sm):
    cid = lax.axis_index("c")
    sid = lax.axis_index("s")

    def zbody(i, _):
        cnt_slice[pl.ds(i * 16, 16)] = _Z16()
        return 0
    lax.fori_loop(0, RED // 16, zbody, 0)

    lo = sid * RED
    for blk in range(N_EDGES // BLK):
        base = blk * BLK
        pltpu.sync_copy(et_hbm.at[pl.ds(base, BLK)], tbuf)
        pltpu.sync_copy(dst_hbm.at[pl.ds(base, BLK)], dbuf)

        def cbody(i, _):
            t = tbuf[pl.ds(i * 16, 16)]
            d = dbuf[pl.ds(i * 16, 16)]
            rel = d * N_REL + t - lo
            m = (rel >= 0) & (rel < RED)
            plsc.addupdate_scatter(cnt_slice, [rel], _O16(), mask=m)
            return 0
        lax.fori_loop(0, BLK // 16, cbody, 0)

    pltpu.sync_copy(cnt_slice, fin_sm.at[pl.ds(lo, RED)])
    plsc.subcore_barrier()

    pltpu.sync_copy(fin_sm, cnt_full)
    ebase = (cid * NS + sid) * E_TILE
    for blk in range(E_TILE // BLK):
        base = ebase + blk * BLK
        pltpu.sync_copy(et_hbm.at[pl.ds(base, BLK)], tbuf)
        pltpu.sync_copy(dst_hbm.at[pl.ds(base, BLK)], dbuf)

        def sbody(i, _):
            t = tbuf[pl.ds(i * 16, 16)]
            d = dbuf[pl.ds(i * 16, 16)]
            c = plsc.load_gather(cnt_full, [d * N_REL + t])
            sbuf[pl.ds(i * 16, 16)] = 1.0 / jnp.maximum(c, 1.0)
            return 0
        lax.fori_loop(0, BLK // 16, sbody, 0)
        pltpu.sync_copy(sbuf, s_hbm.at[pl.ds(base, BLK)])


def _sc_scales(et, dst):
    return pl.kernel(
        _scales_body,
        out_type=jax.ShapeDtypeStruct((N_EDGES,), jnp.float32),
        mesh=_mesh,
        compiler_params=_sc_params,
        scratch_types=[
            pltpu.VMEM((KEYS_PAD,), jnp.float32),
            pltpu.VMEM((RED,), jnp.float32),
            pltpu.VMEM((BLK,), jnp.int32),
            pltpu.VMEM((BLK,), jnp.int32),
            pltpu.VMEM((BLK,), jnp.float32),
            pltpu.VMEM_SHARED((KEYS_PAD,), jnp.float32),
        ],
    )(et, dst)


def _scatter_body(D, y_hbm, src_hbm, et_hbm, dst_hbm, s_hbm, out_hbm,
                  srcb, etb, dstb, sb, gi_v, di_v, rows_v, acc_sm):
    cid = lax.axis_index("c")
    sid = lax.axis_index("s")
    wid = cid * NS + sid
    nq = D // 16

    def zb(i, _):
        for q in range(nq):
            rows_v[i, pl.ds(q * 16, 16)] = _Z16()
        return 0
    lax.fori_loop(0, CHUNK, zb, 0)
    rbase = sid * ROWS_TILE
    for q in range(ROWS_TILE // CHUNK):
        pltpu.sync_copy(rows_v, acc_sm.at[pl.ds(rbase + q * CHUNK, CHUNK)])
    rem = ROWS_TILE % CHUNK
    if rem:
        pltpu.sync_copy(rows_v.at[pl.ds(0, rem)],
                        acc_sm.at[pl.ds(rbase + ROWS_TILE - rem, rem)])
    plsc.subcore_barrier()

    ebase = wid * E_TILE
    for blk in range(E_TILE // BLK):
        bbase = ebase + blk * BLK
        pltpu.sync_copy(src_hbm.at[pl.ds(bbase, BLK)], srcb)
        pltpu.sync_copy(et_hbm.at[pl.ds(bbase, BLK)], etb)
        pltpu.sync_copy(dst_hbm.at[pl.ds(bbase, BLK)], dstb)
        pltpu.sync_copy(s_hbm.at[pl.ds(bbase, BLK)], sb)

        def chunk_body(c, _):
            cb = c * CHUNK
            for j in range(CHUNK // 16):
                t = etb[pl.ds(cb + j * 16, 16)]
                sr = srcb[pl.ds(cb + j * 16, 16)]
                gi_v[pl.ds(j * 16, 16)] = sr * N_REL + t
                di_v[pl.ds(j * 16, 16)] = dstb[pl.ds(cb + j * 16, 16)]
            pltpu.sync_copy(y_hbm.at[gi_v], rows_v)

            def rbody(j, _):
                sj = plsc.load_gather(sb, [jnp.full((16,), cb + j, jnp.int32)])
                for q in range(nq):
                    rows_v[j, pl.ds(q * 16, 16)] = (
                        rows_v[j, pl.ds(q * 16, 16)] * sj)
                return 0
            lax.fori_loop(0, CHUNK, rbody, 0)
            pltpu.sync_copy(rows_v, acc_sm.at[di_v], add=True)
            return 0
        lax.fori_loop(0, BLK // CHUNK, chunk_body, 0)

    plsc.subcore_barrier()
    ob = pl.multiple_of(sid * 624, 8)

    @pl.when(sid < NS - 1)
    def _():
        pltpu.sync_copy(acc_sm.at[pl.ds(ob, 624)],
                        out_hbm.at[cid, pl.ds(ob, 624)])

    @pl.when(sid == NS - 1)
    def _():
        pltpu.sync_copy(acc_sm.at[pl.ds(ob, 640)],
                        out_hbm.at[cid, pl.ds(ob, 640)])


def _sc_scatter(y, src, et, dst, s, D):
    return pl.kernel(
        functools.partial(_scatter_body, D),
        out_type=jax.ShapeDtypeStruct((NC, N_NODES, D), jnp.float32),
        mesh=_mesh,
        compiler_params=_sc_params,
        scratch_types=[
            pltpu.VMEM((BLK,), jnp.int32),
            pltpu.VMEM((BLK,), jnp.int32),
            pltpu.VMEM((BLK,), jnp.int32),
            pltpu.VMEM((BLK,), jnp.float32),
            pltpu.VMEM((CHUNK,), jnp.int32),
            pltpu.VMEM((CHUNK,), jnp.int32),
            pltpu.VMEM((CHUNK, D), jnp.float32),
            pltpu.VMEM_SHARED((N_NODES, D), jnp.float32),
        ],
    )(y, src, et, dst, s)


def _tc1_body(x_ref, w_ref, root_ref, b_ref, y_ref, h0_ref):
    xb = x_ref[...]
    for r in range(N_REL):
        y_ref[:, r, :] = jnp.dot(xb, w_ref[r], preferred_element_type=jnp.float32)
    h0_ref[...] = (jnp.dot(xb, root_ref[...], preferred_element_type=jnp.float32)
                   + b_ref[...])


def _tc_transform1(x, W, root, b, d_in, d_out):
    blk = 400
    grid = (N_NODES // blk,)
    return pl.pallas_call(
        _tc1_body,
        grid=grid,
        in_specs=[
            pl.BlockSpec((blk, d_in), lambda i: (i, 0)),
            pl.BlockSpec((N_REL, d_in, d_out), lambda i: (0, 0, 0)),
            pl.BlockSpec((d_in, d_out), lambda i: (0, 0)),
            pl.BlockSpec((1, d_out), lambda i: (0, 0)),
        ],
        out_specs=[
            pl.BlockSpec((blk, N_REL, d_out), lambda i: (i, 0, 0)),
            pl.BlockSpec((blk, d_out), lambda i: (i, 0)),
        ],
        out_shape=[
            jax.ShapeDtypeStruct((N_NODES, N_REL, d_out), jnp.float32),
            jax.ShapeDtypeStruct((N_NODES, d_out), jnp.float32),
        ],
        compiler_params=pltpu.CompilerParams(
            dimension_semantics=("parallel",)),
    )(x, W, root, b)


def _tc2_body(h0_ref, pa_ref, pb_ref, w_ref, root_ref, b_ref, y_ref, g0_ref):
    h = jnp.maximum(h0_ref[...] + pa_ref[...] + pb_ref[...], 0.0)
    for r in range(N_REL):
        y_ref[:, r, :] = jnp.dot(h, w_ref[r], preferred_element_type=jnp.float32)
    g0_ref[...] = (jnp.dot(h, root_ref[...], preferred_element_type=jnp.float32)
                   + b_ref[...])


def _tc_transform2(h0, pa, pb, W, root, b, d_in, d_w, d_out):
    blk = 400
    grid = (N_NODES // blk,)
    return pl.pallas_call(
        _tc2_body,
        grid=grid,
        in_specs=[
            pl.BlockSpec((blk, d_in), lambda i: (i, 0)),
            pl.BlockSpec((blk, d_in), lambda i: (i, 0)),
            pl.BlockSpec((blk, d_in), lambda i: (i, 0)),
            pl.BlockSpec((N_REL, d_in, d_w), lambda i: (0, 0, 0)),
            pl.BlockSpec((d_in, d_out), lambda i: (0, 0)),
            pl.BlockSpec((1, d_out), lambda i: (0, 0)),
        ],
        out_specs=[
            pl.BlockSpec((blk, N_REL, d_w), lambda i: (i, 0, 0)),
            pl.BlockSpec((blk, d_out), lambda i: (i, 0)),
        ],
        out_shape=[
            jax.ShapeDtypeStruct((N_NODES, N_REL, d_w), jnp.float32),
            jax.ShapeDtypeStruct((N_NODES, d_out), jnp.float32),
        ],
        compiler_params=pltpu.CompilerParams(
            dimension_semantics=("parallel",)),
    )(h0, pa, pb, W, root, b)


def _fin_body(g0_ref, pa_ref, pb_ref, o_ref, acc_ref):
    i = pl.program_id(0)

    @pl.when(i == 0)
    def _():
        acc_ref[...] = jnp.zeros_like(acc_ref)

    hb = g0_ref[...] + pa_ref[...] + pb_ref[...]
    acc_ref[...] += jnp.sum(hb, axis=0, keepdims=True)

    @pl.when(i == pl.num_programs(0) - 1)
    def _():
        g = acc_ref[...] / N_NODES
        m = jnp.max(g)
        lse = jnp.log(jnp.sum(jnp.exp(g - m))) + m
        o_ref[...] = g - lse


def _tc_finish(g0, pa, pb, d):
    blk = 400
    grid = (N_NODES // blk,)
    return pl.pallas_call(
        _fin_body,
        grid=grid,
        in_specs=[
            pl.BlockSpec((blk, d), lambda i: (i, 0)),
            pl.BlockSpec((blk, d), lambda i: (i, 0)),
            pl.BlockSpec((blk, d), lambda i: (i, 0)),
        ],
        out_specs=pl.BlockSpec((1, d), lambda i: (0, 0)),
        out_shape=jax.ShapeDtypeStruct((1, d), jnp.float32),
        scratch_shapes=[pltpu.VMEM((1, d), jnp.float32)],
        compiler_params=pltpu.CompilerParams(
            dimension_semantics=("arbitrary",)),
    )(g0, pa, pb)


def kernel(x, edge_index, edge_type, W1, root1, b1, W2, root2, b2):
    src = edge_index[0]
    dst = edge_index[1]
    et = edge_type

    s = _sc_scales(et, dst)

    y1, h0 = _tc_transform1(x, W1, root1, b1.reshape(1, D_HID), D_IN, D_HID)
    p1 = _sc_scatter(y1.reshape(N_NODES * N_REL, D_HID), src, et, dst, s, D_HID)

    W2p = jnp.pad(W2, ((0, 0), (0, 0), (0, D_HID - D_OUT)))
    y2, g0 = _tc_transform2(h0, p1[0], p1[1], W2p, root2, b2.reshape(1, D_OUT),
                            D_HID, D_HID, D_OUT)
    p2 = _sc_scatter(y2.reshape(N_NODES * N_REL, D_HID), src, et, dst, s, D_HID)

    return _tc_finish(g0, p2[0, :, :D_OUT], p2[1, :, :D_OUT], D_OUT)


# speedup vs baseline: 25.9144x; 2.1098x over previous
import functools

import jax
import jax.numpy as jnp
from jax import lax
from jax.experimental import pallas as pl
from jax.experimental.pallas import tpu as pltpu
from jax.experimental.pallas import tpu_sc as plsc

N_NODES = 10000
N_EDGES = 320000
D_IN = 128
D_HID = 128
D_OUT = 64
N_REL = 8

NC, NS = 2, 16
NW = NC * NS
E_TILE = N_EDGES // NW
E_CNT = N_EDGES // NS
KEYS = N_NODES * N_REL
KEYS_PAD = 81920
RED = KEYS_PAD // NS
CHUNK = 80
BLK = 2000
ROWS_TILE = N_NODES // NS

_mesh = plsc.VectorSubcoreMesh(core_axis_name="c", subcore_axis_name="s")
_sc_params = pltpu.CompilerParams(needs_layout_passes=False)

_Z16 = functools.partial(jnp.zeros, (16,), jnp.float32)
_O16 = functools.partial(jnp.ones, (16,), jnp.float32)


HROWS = KEYS_PAD // 128


def _scales_body(et_hbm, dst_hbm, s_hbm,
                 hist2, tbuf, dbuf, sbuf, zbuf, ridx, acc_sm):
    cid = lax.axis_index("c")
    sid = lax.axis_index("s")

    def zh(i, _):
        for q in range(8):
            hist2[i, pl.ds(q * 16, 16)] = _Z16()
        return 0
    lax.fori_loop(0, HROWS, zh, 0)
    def zz(i, _):
        for q in range(8):
            zbuf[i, pl.ds(q * 16, 16)] = _Z16()
        return 0
    lax.fori_loop(0, HROWS // NS, zz, 0)
    pltpu.sync_copy(zbuf, acc_sm.at[pl.ds(sid * (HROWS // NS), HROWS // NS)])
    plsc.subcore_barrier()

    for blk in range(E_CNT // BLK):
        base = sid * E_CNT + blk * BLK
        pltpu.sync_copy(et_hbm.at[pl.ds(base, BLK)], tbuf)
        pltpu.sync_copy(dst_hbm.at[pl.ds(base, BLK)], dbuf)

        def cbody(i, _):
            t = tbuf[pl.ds(i * 16, 16)]
            d = dbuf[pl.ds(i * 16, 16)]
            key = d * N_REL + t
            plsc.addupdate_scatter(
                hist2, [key >> 7, key & 127], _O16())
            return 0
        lax.fori_loop(0, BLK // 16, cbody, 0)

    for k in range(HROWS // 128):
        for j in range(8):
            ridx[pl.ds(j * 16, 16)] = (
                lax.iota(jnp.int32, 16) + (k * 128 + j * 16))
        pltpu.sync_copy(hist2.at[pl.ds(k * 128, 128)],
                        acc_sm.at[ridx], add=True)
    plsc.subcore_barrier()

    pltpu.sync_copy(acc_sm, hist2)
    ebase = (cid * NS + sid) * E_TILE
    for blk in range(E_TILE // BLK):
        base = ebase + blk * BLK
        pltpu.sync_copy(et_hbm.at[pl.ds(base, BLK)], tbuf)
        pltpu.sync_copy(dst_hbm.at[pl.ds(base, BLK)], dbuf)

        def sbody(i, _):
            t = tbuf[pl.ds(i * 16, 16)]
            d = dbuf[pl.ds(i * 16, 16)]
            key = d * N_REL + t
            c = plsc.load_gather(hist2, [key >> 7, key & 127])
            sbuf[pl.ds(i * 16, 16)] = 1.0 / jnp.maximum(c, 1.0)
            return 0
        lax.fori_loop(0, BLK // 16, sbody, 0)
        pltpu.sync_copy(sbuf, s_hbm.at[pl.ds(base, BLK)])


def _sc_scales(et, dst):
    return pl.kernel(
        _scales_body,
        out_type=jax.ShapeDtypeStruct((N_EDGES,), jnp.float32),
        mesh=_mesh,
        compiler_params=_sc_params,
        scratch_types=[
            pltpu.VMEM((HROWS, 128), jnp.float32),
            pltpu.VMEM((BLK,), jnp.int32),
            pltpu.VMEM((BLK,), jnp.int32),
            pltpu.VMEM((BLK,), jnp.float32),
            pltpu.VMEM((HROWS // NS, 128), jnp.float32),
            pltpu.VMEM((128,), jnp.int32),
            pltpu.VMEM_SHARED((HROWS, 128), jnp.float32),
        ],
    )(et, dst)


def _scatter_body(D, y_hbm, src_hbm, et_hbm, dst_hbm, s_hbm, out_hbm,
                  srcb, etb, dstb, sb, gi_v, di_v, rows_v,
                  gi2_v, di2_v, rows2_v, sem_a, sem_b, acc_sm):
    cid = lax.axis_index("c")
    sid = lax.axis_index("s")
    wid = cid * NS + sid
    nq = D // 16

    def zb(i, _):
        for q in range(nq):
            rows_v[i, pl.ds(q * 16, 16)] = _Z16()
        return 0
    lax.fori_loop(0, CHUNK, zb, 0)
    rbase = sid * ROWS_TILE
    for q in range(ROWS_TILE // CHUNK):
        pltpu.sync_copy(rows_v, acc_sm.at[pl.ds(rbase + q * CHUNK, CHUNK)])
    rem = ROWS_TILE % CHUNK
    if rem:
        pltpu.sync_copy(rows_v.at[pl.ds(0, rem)],
                        acc_sm.at[pl.ds(rbase + ROWS_TILE - rem, rem)])
    plsc.subcore_barrier()

    ebase = wid * E_TILE
    bufs = ((gi_v, di_v, rows_v, sem_a), (gi2_v, di2_v, rows2_v, sem_b))
    nch = BLK // CHUNK

    def build_and_start(c, buf):
        gi, di, rows, sem = buf
        cb = c * CHUNK
        for j in range(CHUNK // 16):
            t = etb[pl.ds(cb + j * 16, 16)]
            sr = srcb[pl.ds(cb + j * 16, 16)]
            gi[pl.ds(j * 16, 16)] = sr * N_REL + t
            di[pl.ds(j * 16, 16)] = dstb[pl.ds(cb + j * 16, 16)]
        pltpu.async_copy(y_hbm.at[gi], rows, sem)

    def consume(c, buf):
        gi, di, rows, sem = buf
        cb = c * CHUNK
        pltpu.make_async_copy(y_hbm.at[gi], rows, sem).wait()

        def rbody(j, _):
            sj = plsc.load_gather(sb, [jnp.full((16,), cb + j, jnp.int32)])
            for q in range(nq):
                rows[j, pl.ds(q * 16, 16)] = rows[j, pl.ds(q * 16, 16)] * sj
            return 0
        lax.fori_loop(0, CHUNK, rbody, 0)
        pltpu.sync_copy(rows, acc_sm.at[di], add=True)

    for blk in range(E_TILE // BLK):
        bbase = ebase + blk * BLK
        pltpu.sync_copy(src_hbm.at[pl.ds(bbase, BLK)], srcb)
        pltpu.sync_copy(et_hbm.at[pl.ds(bbase, BLK)], etb)
        pltpu.sync_copy(dst_hbm.at[pl.ds(bbase, BLK)], dstb)
        pltpu.sync_copy(s_hbm.at[pl.ds(bbase, BLK)], sb)

        build_and_start(0, bufs[0])

        def chunk_body(c, _):
            nxt = c + 1
            even = (c % 2) == 0

            @pl.when((nxt < nch) & even)
            def _():
                build_and_start(nxt, bufs[1])

            @pl.when((nxt < nch) & (~even))
            def _():
                build_and_start(nxt, bufs[0])

            @pl.when(even)
            def _():
                consume(c, bufs[0])

            @pl.when(~even)
            def _():
                consume(c, bufs[1])
            return 0
        lax.fori_loop(0, nch, chunk_body, 0)

    plsc.subcore_barrier()
    ob = pl.multiple_of(sid * 624, 8)

    @pl.when(sid < NS - 1)
    def _():
        pltpu.sync_copy(acc_sm.at[pl.ds(ob, 624)],
                        out_hbm.at[cid, pl.ds(ob, 624)])

    @pl.when(sid == NS - 1)
    def _():
        pltpu.sync_copy(acc_sm.at[pl.ds(ob, 640)],
                        out_hbm.at[cid, pl.ds(ob, 640)])


def _sc_scatter(y, src, et, dst, s, D):
    return pl.kernel(
        functools.partial(_scatter_body, D),
        out_type=jax.ShapeDtypeStruct((NC, N_NODES, D), jnp.float32),
        mesh=_mesh,
        compiler_params=_sc_params,
        scratch_types=[
            pltpu.VMEM((BLK,), jnp.int32),
            pltpu.VMEM((BLK,), jnp.int32),
            pltpu.VMEM((BLK,), jnp.int32),
            pltpu.VMEM((BLK,), jnp.float32),
            pltpu.VMEM((CHUNK,), jnp.int32),
            pltpu.VMEM((CHUNK,), jnp.int32),
            pltpu.VMEM((CHUNK, D), jnp.float32),
            pltpu.VMEM((CHUNK,), jnp.int32),
            pltpu.VMEM((CHUNK,), jnp.int32),
            pltpu.VMEM((CHUNK, D), jnp.float32),
            pltpu.SemaphoreType.DMA,
            pltpu.SemaphoreType.DMA,
            pltpu.VMEM_SHARED((N_NODES, D), jnp.float32),
        ],
    )(y, src, et, dst, s)


def _tc1_body(x_ref, w_ref, root_ref, b_ref, y_ref, h0_ref):
    xb = x_ref[...]
    for r in range(N_REL):
        y_ref[:, r, :] = jnp.dot(xb, w_ref[r], preferred_element_type=jnp.float32)
    h0_ref[...] = (jnp.dot(xb, root_ref[...], preferred_element_type=jnp.float32)
                   + b_ref[...])


def _tc_transform1(x, W, root, b, d_in, d_out):
    blk = 400
    grid = (N_NODES // blk,)
    return pl.pallas_call(
        _tc1_body,
        grid=grid,
        in_specs=[
            pl.BlockSpec((blk, d_in), lambda i: (i, 0)),
            pl.BlockSpec((N_REL, d_in, d_out), lambda i: (0, 0, 0)),
            pl.BlockSpec((d_in, d_out), lambda i: (0, 0)),
            pl.BlockSpec((1, d_out), lambda i: (0, 0)),
        ],
        out_specs=[
            pl.BlockSpec((blk, N_REL, d_out), lambda i: (i, 0, 0)),
            pl.BlockSpec((blk, d_out), lambda i: (i, 0)),
        ],
        out_shape=[
            jax.ShapeDtypeStruct((N_NODES, N_REL, d_out), jnp.float32),
            jax.ShapeDtypeStruct((N_NODES, d_out), jnp.float32),
        ],
        compiler_params=pltpu.CompilerParams(
            dimension_semantics=("parallel",)),
    )(x, W, root, b)


def _tc2_body(h0_ref, pa_ref, pb_ref, w_ref, root_ref, b_ref, y_ref, g0_ref):
    h = jnp.maximum(h0_ref[...] + pa_ref[...] + pb_ref[...], 0.0)
    for r in range(N_REL):
        y_ref[:, r, :] = jnp.dot(h, w_ref[r], preferred_element_type=jnp.float32)
    g0_ref[...] = (jnp.dot(h, root_ref[...], preferred_element_type=jnp.float32)
                   + b_ref[...])


def _tc_transform2(h0, pa, pb, W, root, b, d_in, d_w, d_out):
    blk = 400
    grid = (N_NODES // blk,)
    return pl.pallas_call(
        _tc2_body,
        grid=grid,
        in_specs=[
            pl.BlockSpec((blk, d_in), lambda i: (i, 0)),
            pl.BlockSpec((blk, d_in), lambda i: (i, 0)),
            pl.BlockSpec((blk, d_in), lambda i: (i, 0)),
            pl.BlockSpec((N_REL, d_in, d_w), lambda i: (0, 0, 0)),
            pl.BlockSpec((d_in, d_out), lambda i: (0, 0)),
            pl.BlockSpec((1, d_out), lambda i: (0, 0)),
        ],
        out_specs=[
            pl.BlockSpec((blk, N_REL, d_w), lambda i: (i, 0, 0)),
            pl.BlockSpec((blk, d_out), lambda i: (i, 0)),
        ],
        out_shape=[
            jax.ShapeDtypeStruct((N_NODES, N_REL, d_w), jnp.float32),
            jax.ShapeDtypeStruct((N_NODES, d_out), jnp.float32),
        ],
        compiler_params=pltpu.CompilerParams(
            dimension_semantics=("parallel",)),
    )(h0, pa, pb, W, root, b)


def _fin_body(g0_ref, pa_ref, pb_ref, o_ref, acc_ref):
    i = pl.program_id(0)

    @pl.when(i == 0)
    def _():
        acc_ref[...] = jnp.zeros_like(acc_ref)

    hb = g0_ref[...] + pa_ref[...] + pb_ref[...]
    acc_ref[...] += jnp.sum(hb, axis=0, keepdims=True)

    @pl.when(i == pl.num_programs(0) - 1)
    def _():
        g = acc_ref[...] / N_NODES
        m = jnp.max(g)
        lse = jnp.log(jnp.sum(jnp.exp(g - m))) + m
        o_ref[...] = g - lse


def _tc_finish(g0, pa, pb, d):
    blk = 400
    grid = (N_NODES // blk,)
    return pl.pallas_call(
        _fin_body,
        grid=grid,
        in_specs=[
            pl.BlockSpec((blk, d), lambda i: (i, 0)),
            pl.BlockSpec((blk, d), lambda i: (i, 0)),
            pl.BlockSpec((blk, d), lambda i: (i, 0)),
        ],
        out_specs=pl.BlockSpec((1, d), lambda i: (0, 0)),
        out_shape=jax.ShapeDtypeStruct((1, d), jnp.float32),
        scratch_shapes=[pltpu.VMEM((1, d), jnp.float32)],
        compiler_params=pltpu.CompilerParams(
            dimension_semantics=("arbitrary",)),
    )(g0, pa, pb)


def kernel(x, edge_index, edge_type, W1, root1, b1, W2, root2, b2):
    src = edge_index[0]
    dst = edge_index[1]
    et = edge_type

    s = _sc_scales(et, dst)

    y1, h0 = _tc_transform1(x, W1, root1, b1.reshape(1, D_HID), D_IN, D_HID)
    p1 = _sc_scatter(y1.reshape(N_NODES * N_REL, D_HID), src, et, dst, s, D_HID)

    W2p = jnp.pad(W2, ((0, 0), (0, 0), (0, D_HID - D_OUT)))
    y2, g0 = _tc_transform2(h0, p1[0], p1[1], W2p, root2, b2.reshape(1, D_OUT),
                            D_HID, D_HID, D_OUT)
    p2 = _sc_scatter(y2.reshape(N_NODES * N_REL, D_HID), src, et, dst, s, D_HID)

    return _tc_finish(g0, p2[0, :, :D_OUT], p2[1, :, :D_OUT], D_OUT)


# speedup vs baseline: 28.8511x; 1.1133x over previous
import functools

import jax
import jax.numpy as jnp
from jax import lax
from jax.experimental import pallas as pl
from jax.experimental.pallas import tpu as pltpu
from jax.experimental.pallas import tpu_sc as plsc

N_NODES = 10000
N_EDGES = 320000
D_IN = 128
D_HID = 128
D_OUT = 64
N_REL = 8

NC, NS = 2, 16
NW = NC * NS
E_TILE = N_EDGES // NW
E_CNT = N_EDGES // NS
KEYS = N_NODES * N_REL
KEYS_PAD = 81920
RED = KEYS_PAD // NS
CHUNK = 80
BLK = 2000
ROWS_TILE = N_NODES // NS

_mesh = plsc.VectorSubcoreMesh(core_axis_name="c", subcore_axis_name="s")
_sc_params = pltpu.CompilerParams(needs_layout_passes=False)

_Z16 = functools.partial(jnp.zeros, (16,), jnp.float32)
_O16 = functools.partial(jnp.ones, (16,), jnp.float32)


HROWS = KEYS_PAD // 128


def _scales_body(et_hbm, dst_hbm, s_hbm,
                 hist2, tbuf, dbuf, sbuf, zbuf, ridx, acc_sm):
    cid = lax.axis_index("c")
    sid = lax.axis_index("s")

    def zh(i, _):
        for q in range(8):
            hist2[i, pl.ds(q * 16, 16)] = _Z16()
        return 0
    lax.fori_loop(0, HROWS, zh, 0)
    def zz(i, _):
        for q in range(8):
            zbuf[i, pl.ds(q * 16, 16)] = _Z16()
        return 0
    lax.fori_loop(0, HROWS // NS, zz, 0)
    pltpu.sync_copy(zbuf, acc_sm.at[pl.ds(sid * (HROWS // NS), HROWS // NS)])
    plsc.subcore_barrier()

    for blk in range(E_CNT // BLK):
        base = sid * E_CNT + blk * BLK
        pltpu.sync_copy(et_hbm.at[pl.ds(base, BLK)], tbuf)
        pltpu.sync_copy(dst_hbm.at[pl.ds(base, BLK)], dbuf)

        def cbody(i, _):
            t = tbuf[pl.ds(i * 16, 16)]
            d = dbuf[pl.ds(i * 16, 16)]
            key = d * N_REL + t
            plsc.addupdate_scatter(
                hist2, [key >> 7, key & 127], _O16())
            return 0
        lax.fori_loop(0, BLK // 16, cbody, 0)

    for k in range(HROWS // 128):
        for j in range(8):
            ridx[pl.ds(j * 16, 16)] = (
                lax.iota(jnp.int32, 16) + (k * 128 + j * 16))
        pltpu.sync_copy(hist2.at[pl.ds(k * 128, 128)],
                        acc_sm.at[ridx], add=True)
    plsc.subcore_barrier()

    pltpu.sync_copy(acc_sm, hist2)
    ebase = (cid * NS + sid) * E_TILE
    for blk in range(E_TILE // BLK):
        base = ebase + blk * BLK
        pltpu.sync_copy(et_hbm.at[pl.ds(base, BLK)], tbuf)
        pltpu.sync_copy(dst_hbm.at[pl.ds(base, BLK)], dbuf)

        def sbody(i, _):
            t = tbuf[pl.ds(i * 16, 16)]
            d = dbuf[pl.ds(i * 16, 16)]
            key = d * N_REL + t
            c = plsc.load_gather(hist2, [key >> 7, key & 127])
            sbuf[pl.ds(i * 16, 16)] = 1.0 / jnp.maximum(c, 1.0)
            return 0
        lax.fori_loop(0, BLK // 16, sbody, 0)
        pltpu.sync_copy(sbuf, s_hbm.at[pl.ds(base, BLK)])


def _sc_scales(et, dst):
    return pl.kernel(
        _scales_body,
        out_type=jax.ShapeDtypeStruct((N_EDGES,), jnp.float32),
        mesh=_mesh,
        compiler_params=_sc_params,
        scratch_types=[
            pltpu.VMEM((HROWS, 128), jnp.float32),
            pltpu.VMEM((BLK,), jnp.int32),
            pltpu.VMEM((BLK,), jnp.int32),
            pltpu.VMEM((BLK,), jnp.float32),
            pltpu.VMEM((HROWS // NS, 128), jnp.float32),
            pltpu.VMEM((128,), jnp.int32),
            pltpu.VMEM_SHARED((HROWS, 128), jnp.float32),
        ],
    )(et, dst)


def _scatter_body(D, y_hbm, src_hbm, et_hbm, dst_hbm, s_hbm, out_hbm,
                  srcb, etb, dstb, sb,
                  gi_v, di_v, rows_v, gi2_v, di2_v, rows2_v,
                  gi3_v, di3_v, rows3_v, gi4_v, di4_v, rows4_v,
                  gsem_a, ssem_a, gsem_b, ssem_b,
                  gsem_c, ssem_c, gsem_d, ssem_d, acc_sm):
    cid = lax.axis_index("c")
    sid = lax.axis_index("s")
    wid = cid * NS + sid
    nq = D // 16

    def zb(i, _):
        for q in range(nq):
            rows_v[i, pl.ds(q * 16, 16)] = _Z16()
        return 0
    lax.fori_loop(0, CHUNK, zb, 0)
    rbase = sid * ROWS_TILE
    for q in range(ROWS_TILE // CHUNK):
        pltpu.sync_copy(rows_v, acc_sm.at[pl.ds(rbase + q * CHUNK, CHUNK)])
    rem = ROWS_TILE % CHUNK
    if rem:
        pltpu.sync_copy(rows_v.at[pl.ds(0, rem)],
                        acc_sm.at[pl.ds(rbase + ROWS_TILE - rem, rem)])
    plsc.subcore_barrier()

    ebase = wid * E_TILE
    bufs = ((gi_v, di_v, rows_v, gsem_a, ssem_a),
            (gi2_v, di2_v, rows2_v, gsem_b, ssem_b),
            (gi3_v, di3_v, rows3_v, gsem_c, ssem_c),
            (gi4_v, di4_v, rows4_v, gsem_d, ssem_d))
    NB = 4
    nch = BLK // CHUNK

    def build_and_start(c, buf, drain):
        gi, di, rows, gsem, ssem = buf
        if drain:
            pltpu.make_async_copy(rows, acc_sm.at[di], ssem).wait()
        cb = c * CHUNK
        for j in range(CHUNK // 16):
            t = etb[pl.ds(cb + j * 16, 16)]
            sr = srcb[pl.ds(cb + j * 16, 16)]
            gi[pl.ds(j * 16, 16)] = sr * N_REL + t
            di[pl.ds(j * 16, 16)] = dstb[pl.ds(cb + j * 16, 16)]
        pltpu.async_copy(y_hbm.at[gi], rows, gsem)

    def consume(c, buf):
        gi, di, rows, gsem, ssem = buf
        cb = c * CHUNK
        pltpu.make_async_copy(y_hbm.at[gi], rows, gsem).wait()

        def rbody(j, _):
            sj = plsc.load_gather(sb, [jnp.full((16,), cb + j, jnp.int32)])
            for q in range(nq):
                rows[j, pl.ds(q * 16, 16)] = rows[j, pl.ds(q * 16, 16)] * sj
            return 0
        lax.fori_loop(0, CHUNK, rbody, 0)
        pltpu.async_copy(rows, acc_sm.at[di], ssem, add=True)

    for blk in range(E_TILE // BLK):
        bbase = ebase + blk * BLK
        pltpu.sync_copy(src_hbm.at[pl.ds(bbase, BLK)], srcb)
        pltpu.sync_copy(et_hbm.at[pl.ds(bbase, BLK)], etb)
        pltpu.sync_copy(dst_hbm.at[pl.ds(bbase, BLK)], dstb)
        pltpu.sync_copy(s_hbm.at[pl.ds(bbase, BLK)], sb)

        build_and_start(0, bufs[0], drain=False)
        build_and_start(1, bufs[1], drain=False)

        def chunk_body(c, _):
            for k in range(NB):
                @pl.when(c % NB == k)
                def _(k=k, c=c):
                    consume(c, bufs[k])

            nxt = c + 2
            for k in range(NB):
                @pl.when((nxt < nch) & (nxt % NB == k) & (nxt >= NB))
                def _(k=k, nxt=nxt):
                    build_and_start(nxt, bufs[k], drain=True)

                @pl.when((nxt < nch) & (nxt % NB == k) & (nxt < NB))
                def _(k=k, nxt=nxt):
                    build_and_start(nxt, bufs[k], drain=False)
            return 0
        lax.fori_loop(0, nch, chunk_body, 0)

        for k in range(NB):
            gi, di, rows, gsem, ssem = bufs[k]
            pltpu.make_async_copy(rows, acc_sm.at[di], ssem).wait()

    plsc.subcore_barrier()
    ob = pl.multiple_of(sid * 624, 8)

    @pl.when(sid < NS - 1)
    def _():
        pltpu.sync_copy(acc_sm.at[pl.ds(ob, 624)],
                        out_hbm.at[cid, pl.ds(ob, 624)])

    @pl.when(sid == NS - 1)
    def _():
        pltpu.sync_copy(acc_sm.at[pl.ds(ob, 640)],
                        out_hbm.at[cid, pl.ds(ob, 640)])


def _sc_scatter(y, src, et, dst, s, D):
    return pl.kernel(
        functools.partial(_scatter_body, D),
        out_type=jax.ShapeDtypeStruct((NC, N_NODES, D), jnp.float32),
        mesh=_mesh,
        compiler_params=_sc_params,
        scratch_types=[
            pltpu.VMEM((BLK,), jnp.int32),
            pltpu.VMEM((BLK,), jnp.int32),
            pltpu.VMEM((BLK,), jnp.int32),
            pltpu.VMEM((BLK,), jnp.float32),
            pltpu.VMEM((CHUNK,), jnp.int32),
            pltpu.VMEM((CHUNK,), jnp.int32),
            pltpu.VMEM((CHUNK, D), jnp.float32),
            pltpu.VMEM((CHUNK,), jnp.int32),
            pltpu.VMEM((CHUNK,), jnp.int32),
            pltpu.VMEM((CHUNK, D), jnp.float32),
            pltpu.VMEM((CHUNK,), jnp.int32),
            pltpu.VMEM((CHUNK,), jnp.int32),
            pltpu.VMEM((CHUNK, D), jnp.float32),
            pltpu.VMEM((CHUNK,), jnp.int32),
            pltpu.VMEM((CHUNK,), jnp.int32),
            pltpu.VMEM((CHUNK, D), jnp.float32),
            pltpu.SemaphoreType.DMA,
            pltpu.SemaphoreType.DMA,
            pltpu.SemaphoreType.DMA,
            pltpu.SemaphoreType.DMA,
            pltpu.SemaphoreType.DMA,
            pltpu.SemaphoreType.DMA,
            pltpu.SemaphoreType.DMA,
            pltpu.SemaphoreType.DMA,
            pltpu.VMEM_SHARED((N_NODES, D), jnp.float32),
        ],
    )(y, src, et, dst, s)


def _tc1_body(x_ref, w_ref, root_ref, b_ref, y_ref, h0_ref):
    xb = x_ref[...]
    for r in range(N_REL):
        y_ref[:, r, :] = jnp.dot(xb, w_ref[r], preferred_element_type=jnp.float32)
    h0_ref[...] = (jnp.dot(xb, root_ref[...], preferred_element_type=jnp.float32)
                   + b_ref[...])


def _tc_transform1(x, W, root, b, d_in, d_out):
    blk = 400
    grid = (N_NODES // blk,)
    return pl.pallas_call(
        _tc1_body,
        grid=grid,
        in_specs=[
            pl.BlockSpec((blk, d_in), lambda i: (i, 0)),
            pl.BlockSpec((N_REL, d_in, d_out), lambda i: (0, 0, 0)),
            pl.BlockSpec((d_in, d_out), lambda i: (0, 0)),
            pl.BlockSpec((1, d_out), lambda i: (0, 0)),
        ],
        out_specs=[
            pl.BlockSpec((blk, N_REL, d_out), lambda i: (i, 0, 0)),
            pl.BlockSpec((blk, d_out), lambda i: (i, 0)),
        ],
        out_shape=[
            jax.ShapeDtypeStruct((N_NODES, N_REL, d_out), jnp.float32),
            jax.ShapeDtypeStruct((N_NODES, d_out), jnp.float32),
        ],
        compiler_params=pltpu.CompilerParams(
            dimension_semantics=("parallel",)),
    )(x, W, root, b)


def _tc2_body(h0_ref, pa_ref, pb_ref, w_ref, root_ref, b_ref, y_ref, g0_ref):
    h = jnp.maximum(h0_ref[...] + pa_ref[...] + pb_ref[...], 0.0)
    for r in range(N_REL):
        y_ref[:, r, :] = jnp.dot(h, w_ref[r], preferred_element_type=jnp.float32)
    g0_ref[...] = (jnp.dot(h, root_ref[...], preferred_element_type=jnp.float32)
                   + b_ref[...])


def _tc_transform2(h0, pa, pb, W, root, b, d_in, d_w, d_out):
    blk = 400
    grid = (N_NODES // blk,)
    return pl.pallas_call(
        _tc2_body,
        grid=grid,
        in_specs=[
            pl.BlockSpec((blk, d_in), lambda i: (i, 0)),
            pl.BlockSpec((blk, d_in), lambda i: (i, 0)),
            pl.BlockSpec((blk, d_in), lambda i: (i, 0)),
            pl.BlockSpec((N_REL, d_in, d_w), lambda i: (0, 0, 0)),
            pl.BlockSpec((d_in, d_out), lambda i: (0, 0)),
            pl.BlockSpec((1, d_out), lambda i: (0, 0)),
        ],
        out_specs=[
            pl.BlockSpec((blk, N_REL, d_w), lambda i: (i, 0, 0)),
            pl.BlockSpec((blk, d_out), lambda i: (i, 0)),
        ],
        out_shape=[
            jax.ShapeDtypeStruct((N_NODES, N_REL, d_w), jnp.float32),
            jax.ShapeDtypeStruct((N_NODES, d_out), jnp.float32),
        ],
        compiler_params=pltpu.CompilerParams(
            dimension_semantics=("parallel",)),
    )(h0, pa, pb, W, root, b)


def _fin_body(g0_ref, pa_ref, pb_ref, o_ref, acc_ref):
    i = pl.program_id(0)

    @pl.when(i == 0)
    def _():
        acc_ref[...] = jnp.zeros_like(acc_ref)

    hb = g0_ref[...] + pa_ref[...] + pb_ref[...]
    acc_ref[...] += jnp.sum(hb, axis=0, keepdims=True)

    @pl.when(i == pl.num_programs(0) - 1)
    def _():
        g = acc_ref[...] / N_NODES
        m = jnp.max(g)
        lse = jnp.log(jnp.sum(jnp.exp(g - m))) + m
        o_ref[...] = g - lse


def _tc_finish(g0, pa, pb, d):
    blk = 400
    grid = (N_NODES // blk,)
    return pl.pallas_call(
        _fin_body,
        grid=grid,
        in_specs=[
            pl.BlockSpec((blk, d), lambda i: (i, 0)),
            pl.BlockSpec((blk, d), lambda i: (i, 0)),
            pl.BlockSpec((blk, d), lambda i: (i, 0)),
        ],
        out_specs=pl.BlockSpec((1, d), lambda i: (0, 0)),
        out_shape=jax.ShapeDtypeStruct((1, d), jnp.float32),
        scratch_shapes=[pltpu.VMEM((1, d), jnp.float32)],
        compiler_params=pltpu.CompilerParams(
            dimension_semantics=("arbitrary",)),
    )(g0, pa, pb)


def kernel(x, edge_index, edge_type, W1, root1, b1, W2, root2, b2):
    src = edge_index[0]
    dst = edge_index[1]
    et = edge_type

    s = _sc_scales(et, dst)

    y1, h0 = _tc_transform1(x, W1, root1, b1.reshape(1, D_HID), D_IN, D_HID)
    p1 = _sc_scatter(y1.reshape(N_NODES * N_REL, D_HID), src, et, dst, s, D_HID)

    W2p = jnp.pad(W2, ((0, 0), (0, 0), (0, D_HID - D_OUT)))
    y2, g0 = _tc_transform2(h0, p1[0], p1[1], W2p, root2, b2.reshape(1, D_OUT),
                            D_HID, D_HID, D_OUT)
    p2 = _sc_scatter(y2.reshape(N_NODES * N_REL, D_HID), src, et, dst, s, D_HID)

    return _tc_finish(g0, p2[0, :, :D_OUT], p2[1, :, :D_OUT], D_OUT)


# speedup vs baseline: 32.4694x; 1.1254x over previous
import functools

import jax
import jax.numpy as jnp
from jax import lax
from jax.experimental import pallas as pl
from jax.experimental.pallas import tpu as pltpu
from jax.experimental.pallas import tpu_sc as plsc

N_NODES = 10000
N_EDGES = 320000
D_IN = 128
D_HID = 128
D_OUT = 64
N_REL = 8

NC, NS = 2, 16
NW = NC * NS
E_TILE = N_EDGES // NW
E_CNT = N_EDGES // NS
KEYS = N_NODES * N_REL
KEYS_PAD = 81920
RED = KEYS_PAD // NS
CHUNK = 80
BLK = 2000
ROWS_TILE = N_NODES // NS

_mesh = plsc.VectorSubcoreMesh(core_axis_name="c", subcore_axis_name="s")
_sc_params = pltpu.CompilerParams(needs_layout_passes=False)

_Z16 = functools.partial(jnp.zeros, (16,), jnp.float32)
_O16 = functools.partial(jnp.ones, (16,), jnp.float32)


HROWS = KEYS_PAD // 128


def _scales_body(et_hbm, dst_hbm, s_hbm,
                 hist2, tbuf, dbuf, sbuf, zbuf, ridx, acc_sm):
    cid = lax.axis_index("c")
    sid = lax.axis_index("s")

    def zh(i, _):
        for q in range(8):
            hist2[i, pl.ds(q * 16, 16)] = _Z16()
        return 0
    lax.fori_loop(0, HROWS, zh, 0)
    def zz(i, _):
        for q in range(8):
            zbuf[i, pl.ds(q * 16, 16)] = _Z16()
        return 0
    lax.fori_loop(0, HROWS // NS, zz, 0)
    pltpu.sync_copy(zbuf, acc_sm.at[pl.ds(sid * (HROWS // NS), HROWS // NS)])
    plsc.subcore_barrier()

    for blk in range(E_CNT // BLK):
        base = sid * E_CNT + blk * BLK
        pltpu.sync_copy(et_hbm.at[pl.ds(base, BLK)], tbuf)
        pltpu.sync_copy(dst_hbm.at[pl.ds(base, BLK)], dbuf)

        def cbody(i, _):
            t = tbuf[pl.ds(i * 16, 16)]
            d = dbuf[pl.ds(i * 16, 16)]
            key = d * N_REL + t
            plsc.addupdate_scatter(
                hist2, [key >> 7, key & 127], _O16())
            return 0
        lax.fori_loop(0, BLK // 16, cbody, 0)

    for k in range(HROWS // 128):
        for j in range(8):
            ridx[pl.ds(j * 16, 16)] = (
                lax.iota(jnp.int32, 16) + (k * 128 + j * 16))
        pltpu.sync_copy(hist2.at[pl.ds(k * 128, 128)],
                        acc_sm.at[ridx], add=True)
    plsc.subcore_barrier()

    pltpu.sync_copy(acc_sm, hist2)
    ebase = (cid * NS + sid) * E_TILE
    for blk in range(E_TILE // BLK):
        base = ebase + blk * BLK
        pltpu.sync_copy(et_hbm.at[pl.ds(base, BLK)], tbuf)
        pltpu.sync_copy(dst_hbm.at[pl.ds(base, BLK)], dbuf)

        @plsc.parallel_loop(0, BLK // 16, 1, unroll=4)
        def _(i):
            t = tbuf[pl.ds(i * 16, 16)]
            d = dbuf[pl.ds(i * 16, 16)]
            key = d * N_REL + t
            c = plsc.load_gather(hist2, [key >> 7, key & 127])
            sbuf[pl.ds(i * 16, 16)] = 1.0 / jnp.maximum(c, 1.0)
        pltpu.sync_copy(sbuf, s_hbm.at[pl.ds(base, BLK)])


def _sc_scales(et, dst):
    return pl.kernel(
        _scales_body,
        out_type=jax.ShapeDtypeStruct((N_EDGES,), jnp.float32),
        mesh=_mesh,
        compiler_params=_sc_params,
        scratch_types=[
            pltpu.VMEM((HROWS, 128), jnp.float32),
            pltpu.VMEM((BLK,), jnp.int32),
            pltpu.VMEM((BLK,), jnp.int32),
            pltpu.VMEM((BLK,), jnp.float32),
            pltpu.VMEM((HROWS // NS, 128), jnp.float32),
            pltpu.VMEM((128,), jnp.int32),
            pltpu.VMEM_SHARED((HROWS, 128), jnp.float32),
        ],
    )(et, dst)


def _scatter_body(D, y_hbm, src_hbm, et_hbm, dst_hbm, s_hbm, out_hbm,
                  srcb, etb, dstb, sb,
                  gi_v, di_v, rows_v, gi2_v, di2_v, rows2_v,
                  gi3_v, di3_v, rows3_v, gi4_v, di4_v, rows4_v,
                  gsem_a, ssem_a, gsem_b, ssem_b,
                  gsem_c, ssem_c, gsem_d, ssem_d, acc_sm):
    cid = lax.axis_index("c")
    sid = lax.axis_index("s")
    wid = cid * NS + sid
    nq = D // 16

    def zb(i, _):
        for q in range(nq):
            rows_v[i, pl.ds(q * 16, 16)] = _Z16()
        return 0
    lax.fori_loop(0, CHUNK, zb, 0)
    rbase = sid * ROWS_TILE
    for q in range(ROWS_TILE // CHUNK):
        pltpu.sync_copy(rows_v, acc_sm.at[pl.ds(rbase + q * CHUNK, CHUNK)])
    rem = ROWS_TILE % CHUNK
    if rem:
        pltpu.sync_copy(rows_v.at[pl.ds(0, rem)],
                        acc_sm.at[pl.ds(rbase + ROWS_TILE - rem, rem)])
    plsc.subcore_barrier()

    ebase = wid * E_TILE
    bufs = ((gi_v, di_v, rows_v, gsem_a, ssem_a),
            (gi2_v, di2_v, rows2_v, gsem_b, ssem_b),
            (gi3_v, di3_v, rows3_v, gsem_c, ssem_c),
            (gi4_v, di4_v, rows4_v, gsem_d, ssem_d))
    NB = 4
    nch = BLK // CHUNK

    def build_and_start(c, buf, drain):
        gi, di, rows, gsem, ssem = buf
        if drain:
            pltpu.make_async_copy(rows, acc_sm.at[di], ssem).wait()
        cb = c * CHUNK
        for j in range(CHUNK // 16):
            t = etb[pl.ds(cb + j * 16, 16)]
            sr = srcb[pl.ds(cb + j * 16, 16)]
            gi[pl.ds(j * 16, 16)] = sr * N_REL + t
            di[pl.ds(j * 16, 16)] = dstb[pl.ds(cb + j * 16, 16)]
        pltpu.async_copy(y_hbm.at[gi], rows, gsem)

    def consume(c, buf):
        gi, di, rows, gsem, ssem = buf
        cb = c * CHUNK
        pltpu.make_async_copy(y_hbm.at[gi], rows, gsem).wait()

        @plsc.parallel_loop(0, CHUNK, 1, unroll=4)
        def _(j):
            sj = plsc.load_gather(sb, [jnp.full((16,), cb + j, jnp.int32)])
            for q in range(nq):
                rows[j, pl.ds(q * 16, 16)] = rows[j, pl.ds(q * 16, 16)] * sj
        pltpu.async_copy(rows, acc_sm.at[di], ssem, add=True)

    for blk in range(E_TILE // BLK):
        bbase = ebase + blk * BLK
        pltpu.sync_copy(src_hbm.at[pl.ds(bbase, BLK)], srcb)
        pltpu.sync_copy(et_hbm.at[pl.ds(bbase, BLK)], etb)
        pltpu.sync_copy(dst_hbm.at[pl.ds(bbase, BLK)], dstb)
        pltpu.sync_copy(s_hbm.at[pl.ds(bbase, BLK)], sb)

        build_and_start(0, bufs[0], drain=False)
        build_and_start(1, bufs[1], drain=False)

        def chunk_body(c, _):
            for k in range(NB):
                @pl.when(c % NB == k)
                def _(k=k, c=c):
                    consume(c, bufs[k])

            nxt = c + 2
            for k in range(NB):
                @pl.when((nxt < nch) & (nxt % NB == k) & (nxt >= NB))
                def _(k=k, nxt=nxt):
                    build_and_start(nxt, bufs[k], drain=True)

                @pl.when((nxt < nch) & (nxt % NB == k) & (nxt < NB))
                def _(k=k, nxt=nxt):
                    build_and_start(nxt, bufs[k], drain=False)
            return 0
        lax.fori_loop(0, nch, chunk_body, 0)

        for k in range(NB):
            gi, di, rows, gsem, ssem = bufs[k]
            pltpu.make_async_copy(rows, acc_sm.at[di], ssem).wait()

    plsc.subcore_barrier()
    ob = pl.multiple_of(sid * 624, 8)

    @pl.when(sid < NS - 1)
    def _():
        pltpu.sync_copy(acc_sm.at[pl.ds(ob, 624)],
                        out_hbm.at[cid, pl.ds(ob, 624)])

    @pl.when(sid == NS - 1)
    def _():
        pltpu.sync_copy(acc_sm.at[pl.ds(ob, 640)],
                        out_hbm.at[cid, pl.ds(ob, 640)])


def _sc_scatter(y, src, et, dst, s, D):
    return pl.kernel(
        functools.partial(_scatter_body, D),
        out_type=jax.ShapeDtypeStruct((NC, N_NODES, D), jnp.float32),
        mesh=_mesh,
        compiler_params=_sc_params,
        scratch_types=[
            pltpu.VMEM((BLK,), jnp.int32),
            pltpu.VMEM((BLK,), jnp.int32),
            pltpu.VMEM((BLK,), jnp.int32),
            pltpu.VMEM((BLK,), jnp.float32),
            pltpu.VMEM((CHUNK,), jnp.int32),
            pltpu.VMEM((CHUNK,), jnp.int32),
            pltpu.VMEM((CHUNK, D), jnp.float32),
            pltpu.VMEM((CHUNK,), jnp.int32),
            pltpu.VMEM((CHUNK,), jnp.int32),
            pltpu.VMEM((CHUNK, D), jnp.float32),
            pltpu.VMEM((CHUNK,), jnp.int32),
            pltpu.VMEM((CHUNK,), jnp.int32),
            pltpu.VMEM((CHUNK, D), jnp.float32),
            pltpu.VMEM((CHUNK,), jnp.int32),
            pltpu.VMEM((CHUNK,), jnp.int32),
            pltpu.VMEM((CHUNK, D), jnp.float32),
            pltpu.SemaphoreType.DMA,
            pltpu.SemaphoreType.DMA,
            pltpu.SemaphoreType.DMA,
            pltpu.SemaphoreType.DMA,
            pltpu.SemaphoreType.DMA,
            pltpu.SemaphoreType.DMA,
            pltpu.SemaphoreType.DMA,
            pltpu.SemaphoreType.DMA,
            pltpu.VMEM_SHARED((N_NODES, D), jnp.float32),
        ],
    )(y, src, et, dst, s)


def _tc1_body(x_ref, w_ref, root_ref, b_ref, y_ref, h0_ref):
    xb = x_ref[...]
    for r in range(N_REL):
        y_ref[:, r, :] = jnp.dot(xb, w_ref[r], preferred_element_type=jnp.float32)
    h0_ref[...] = (jnp.dot(xb, root_ref[...], preferred_element_type=jnp.float32)
                   + b_ref[...])


def _tc_transform1(x, W, root, b, d_in, d_out):
    blk = 400
    grid = (N_NODES // blk,)
    return pl.pallas_call(
        _tc1_body,
        grid=grid,
        in_specs=[
            pl.BlockSpec((blk, d_in), lambda i: (i, 0)),
            pl.BlockSpec((N_REL, d_in, d_out), lambda i: (0, 0, 0)),
            pl.BlockSpec((d_in, d_out), lambda i: (0, 0)),
            pl.BlockSpec((1, d_out), lambda i: (0, 0)),
        ],
        out_specs=[
            pl.BlockSpec((blk, N_REL, d_out), lambda i: (i, 0, 0)),
            pl.BlockSpec((blk, d_out), lambda i: (i, 0)),
        ],
        out_shape=[
            jax.ShapeDtypeStruct((N_NODES, N_REL, d_out), jnp.float32),
            jax.ShapeDtypeStruct((N_NODES, d_out), jnp.float32),
        ],
        compiler_params=pltpu.CompilerParams(
            dimension_semantics=("parallel",)),
    )(x, W, root, b)


def _tc2_body(h0_ref, pa_ref, pb_ref, w_ref, root_ref, b_ref, y_ref, g0_ref):
    h = jnp.maximum(h0_ref[...] + pa_ref[...] + pb_ref[...], 0.0)
    for r in range(N_REL):
        y_ref[:, r, :] = jnp.dot(h, w_ref[r], preferred_element_type=jnp.float32)
    g0_ref[...] = (jnp.dot(h, root_ref[...], preferred_element_type=jnp.float32)
                   + b_ref[...])


def _tc_transform2(h0, pa, pb, W, root, b, d_in, d_w, d_out):
    blk = 400
    grid = (N_NODES // blk,)
    return pl.pallas_call(
        _tc2_body,
        grid=grid,
        in_specs=[
            pl.BlockSpec((blk, d_in), lambda i: (i, 0)),
            pl.BlockSpec((blk, d_in), lambda i: (i, 0)),
            pl.BlockSpec((blk, d_in), lambda i: (i, 0)),
            pl.BlockSpec((N_REL, d_in, d_w), lambda i: (0, 0, 0)),
            pl.BlockSpec((d_in, d_out), lambda i: (0, 0)),
            pl.BlockSpec((1, d_out), lambda i: (0, 0)),
        ],
        out_specs=[
            pl.BlockSpec((blk, N_REL, d_w), lambda i: (i, 0, 0)),
            pl.BlockSpec((blk, d_out), lambda i: (i, 0)),
        ],
        out_shape=[
            jax.ShapeDtypeStruct((N_NODES, N_REL, d_w), jnp.float32),
            jax.ShapeDtypeStruct((N_NODES, d_out), jnp.float32),
        ],
        compiler_params=pltpu.CompilerParams(
            dimension_semantics=("parallel",)),
    )(h0, pa, pb, W, root, b)


def _fin_body(g0_ref, pa_ref, pb_ref, o_ref, acc_ref):
    i = pl.program_id(0)

    @pl.when(i == 0)
    def _():
        acc_ref[...] = jnp.zeros_like(acc_ref)

    hb = g0_ref[...] + pa_ref[...] + pb_ref[...]
    acc_ref[...] += jnp.sum(hb, axis=0, keepdims=True)

    @pl.when(i == pl.num_programs(0) - 1)
    def _():
        g = acc_ref[...] / N_NODES
        m = jnp.max(g)
        lse = jnp.log(jnp.sum(jnp.exp(g - m))) + m
        o_ref[...] = g - lse


def _tc_finish(g0, pa, pb, d):
    blk = 400
    grid = (N_NODES // blk,)
    return pl.pallas_call(
        _fin_body,
        grid=grid,
        in_specs=[
            pl.BlockSpec((blk, d), lambda i: (i, 0)),
            pl.BlockSpec((blk, d), lambda i: (i, 0)),
            pl.BlockSpec((blk, d), lambda i: (i, 0)),
        ],
        out_specs=pl.BlockSpec((1, d), lambda i: (0, 0)),
        out_shape=jax.ShapeDtypeStruct((1, d), jnp.float32),
        scratch_shapes=[pltpu.VMEM((1, d), jnp.float32)],
        compiler_params=pltpu.CompilerParams(
            dimension_semantics=("arbitrary",)),
    )(g0, pa, pb)


def kernel(x, edge_index, edge_type, W1, root1, b1, W2, root2, b2):
    src = edge_index[0]
    dst = edge_index[1]
    et = edge_type

    s = _sc_scales(et, dst)

    y1, h0 = _tc_transform1(x, W1, root1, b1.reshape(1, D_HID), D_IN, D_HID)
    p1 = _sc_scatter(y1.reshape(N_NODES * N_REL, D_HID), src, et, dst, s, D_HID)

    W2p = jnp.pad(W2, ((0, 0), (0, 0), (0, D_HID - D_OUT)))
    y2, g0 = _tc_transform2(h0, p1[0], p1[1], W2p, root2, b2.reshape(1, D_OUT),
                            D_HID, D_HID, D_OUT)
    p2 = _sc_scatter(y2.reshape(N_NODES * N_REL, D_HID), src, et, dst, s, D_HID)

    return _tc_finish(g0, p2[0, :, :D_OUT], p2[1, :, :D_OUT], D_OUT)


# speedup vs baseline: 36.0095x; 1.1090x over previous
import functools

import jax
import jax.numpy as jnp
from jax import lax
from jax.experimental import pallas as pl
from jax.experimental.pallas import tpu as pltpu
from jax.experimental.pallas import tpu_sc as plsc

N_NODES = 10000
N_EDGES = 320000
D_IN = 128
D_HID = 128
D_OUT = 64
N_REL = 8

NC, NS = 2, 16
NW = NC * NS
E_TILE = N_EDGES // NW
E_CNT = N_EDGES // NS
KEYS = N_NODES * N_REL
KEYS_PAD = 81920
RED = KEYS_PAD // NS
CHUNK = 80
BLK = 2000
ROWS_TILE = N_NODES // NS

_mesh = plsc.VectorSubcoreMesh(core_axis_name="c", subcore_axis_name="s")
_sc_params = pltpu.CompilerParams(needs_layout_passes=False)

_Z16 = functools.partial(jnp.zeros, (16,), jnp.float32)
_O16 = functools.partial(jnp.ones, (16,), jnp.float32)


HROWS = KEYS_PAD // 128


def _scales_body(et_hbm, dst_hbm, src_hbm, s_hbm, gi_hbm,
                 hist2, tbuf, dbuf, sbuf, srcb, gibuf, zbuf, ridx, acc_sm):
    cid = lax.axis_index("c")
    sid = lax.axis_index("s")

    def zh(i, _):
        for q in range(8):
            hist2[i, pl.ds(q * 16, 16)] = _Z16()
        return 0
    lax.fori_loop(0, HROWS, zh, 0)
    def zz(i, _):
        for q in range(8):
            zbuf[i, pl.ds(q * 16, 16)] = _Z16()
        return 0
    lax.fori_loop(0, HROWS // NS, zz, 0)
    pltpu.sync_copy(zbuf, acc_sm.at[pl.ds(sid * (HROWS // NS), HROWS // NS)])
    plsc.subcore_barrier()

    for blk in range(E_CNT // BLK):
        base = sid * E_CNT + blk * BLK
        pltpu.sync_copy(et_hbm.at[pl.ds(base, BLK)], tbuf)
        pltpu.sync_copy(dst_hbm.at[pl.ds(base, BLK)], dbuf)

        def cbody(i, _):
            t = tbuf[pl.ds(i * 16, 16)]
            d = dbuf[pl.ds(i * 16, 16)]
            key = d * N_REL + t
            plsc.addupdate_scatter(
                hist2, [key >> 7, key & 127], _O16())
            return 0
        lax.fori_loop(0, BLK // 16, cbody, 0)

    for k in range(HROWS // 128):
        for j in range(8):
            ridx[pl.ds(j * 16, 16)] = (
                lax.iota(jnp.int32, 16) + (k * 128 + j * 16))
        pltpu.sync_copy(hist2.at[pl.ds(k * 128, 128)],
                        acc_sm.at[ridx], add=True)
    plsc.subcore_barrier()

    pltpu.sync_copy(acc_sm, hist2)
    ebase = (cid * NS + sid) * E_TILE
    for blk in range(E_TILE // BLK):
        base = ebase + blk * BLK
        pltpu.sync_copy(et_hbm.at[pl.ds(base, BLK)], tbuf)
        pltpu.sync_copy(dst_hbm.at[pl.ds(base, BLK)], dbuf)
        pltpu.sync_copy(src_hbm.at[pl.ds(base, BLK)], srcb)

        @plsc.parallel_loop(0, BLK // 16, 1, unroll=4)
        def _(i):
            t = tbuf[pl.ds(i * 16, 16)]
            d = dbuf[pl.ds(i * 16, 16)]
            sr = srcb[pl.ds(i * 16, 16)]
            key = d * N_REL + t
            c = plsc.load_gather(hist2, [key >> 7, key & 127])
            sbuf[pl.ds(i * 16, 16)] = 1.0 / jnp.maximum(c, 1.0)
            gibuf[pl.ds(i * 16, 16)] = sr * N_REL + t
        pltpu.sync_copy(sbuf, s_hbm.at[pl.ds(base, BLK)])
        pltpu.sync_copy(gibuf, gi_hbm.at[pl.ds(base, BLK)])


def _sc_scales(et, dst, src):
    return pl.kernel(
        _scales_body,
        out_type=[jax.ShapeDtypeStruct((N_EDGES,), jnp.float32),
                  jax.ShapeDtypeStruct((N_EDGES,), jnp.int32)],
        mesh=_mesh,
        compiler_params=_sc_params,
        scratch_types=[
            pltpu.VMEM((HROWS, 128), jnp.float32),
            pltpu.VMEM((BLK,), jnp.int32),
            pltpu.VMEM((BLK,), jnp.int32),
            pltpu.VMEM((BLK,), jnp.float32),
            pltpu.VMEM((BLK,), jnp.int32),
            pltpu.VMEM((BLK,), jnp.int32),
            pltpu.VMEM((HROWS // NS, 128), jnp.float32),
            pltpu.VMEM((128,), jnp.int32),
            pltpu.VMEM_SHARED((HROWS, 128), jnp.float32),
        ],
    )(et, dst, src)


def _scatter_body(D, y_hbm, gi_hbm, di_hbm, s_hbm, out_hbm,
                  gi1, di1, s1, rows1, gi2, di2, s2, rows2,
                  gi3, di3, s3, rows3, gi4, di4, s4, rows4,
                  msem1, gsem1, ssem1, msem2, gsem2, ssem2,
                  msem3, gsem3, ssem3, msem4, gsem4, ssem4, acc_sm):
    cid = lax.axis_index("c")
    sid = lax.axis_index("s")
    wid = cid * NS + sid
    nq = D // 16

    def zb(i, _):
        for q in range(nq):
            rows1[i, pl.ds(q * 16, 16)] = _Z16()
        return 0
    lax.fori_loop(0, CHUNK, zb, 0)
    rbase = sid * ROWS_TILE
    for q in range(ROWS_TILE // CHUNK):
        pltpu.sync_copy(rows1, acc_sm.at[pl.ds(rbase + q * CHUNK, CHUNK)])
    rem = ROWS_TILE % CHUNK
    if rem:
        pltpu.sync_copy(rows1.at[pl.ds(0, rem)],
                        acc_sm.at[pl.ds(rbase + ROWS_TILE - rem, rem)])
    plsc.subcore_barrier()

    ebase = wid * E_TILE
    bufs = ((gi1, di1, s1, rows1, msem1, gsem1, ssem1),
            (gi2, di2, s2, rows2, msem2, gsem2, ssem2),
            (gi3, di3, s3, rows3, msem3, gsem3, ssem3),
            (gi4, di4, s4, rows4, msem4, gsem4, ssem4))
    NB = 4
    NCH = E_TILE // CHUNK

    def meta_issue(c, buf, drain):
        gi, di, sv, rows, msem, gsem, ssem = buf
        if drain:
            pltpu.make_async_copy(rows, acc_sm.at[di], ssem).wait()
        base = ebase + c * CHUNK
        pltpu.async_copy(gi_hbm.at[pl.ds(base, CHUNK)], gi, msem)
        pltpu.async_copy(di_hbm.at[pl.ds(base, CHUNK)], di, msem)
        pltpu.async_copy(s_hbm.at[pl.ds(base, CHUNK)], sv, msem)

    def gather_issue(c, buf):
        gi, di, sv, rows, msem, gsem, ssem = buf
        base = ebase + c * CHUNK
        pltpu.make_async_copy(gi_hbm.at[pl.ds(base, CHUNK)], gi, msem).wait()
        pltpu.make_async_copy(di_hbm.at[pl.ds(base, CHUNK)], di, msem).wait()
        pltpu.make_async_copy(s_hbm.at[pl.ds(base, CHUNK)], sv, msem).wait()
        pltpu.async_copy(y_hbm.at[gi], rows, gsem)

    def consume(buf):
        gi, di, sv, rows, msem, gsem, ssem = buf
        pltpu.make_async_copy(y_hbm.at[gi], rows, gsem).wait()

        @plsc.parallel_loop(0, CHUNK, 1, unroll=4)
        def _(j):
            sj = plsc.load_gather(sv, [jnp.full((16,), j, jnp.int32)])
            for q in range(nq):
                rows[j, pl.ds(q * 16, 16)] = rows[j, pl.ds(q * 16, 16)] * sj
        pltpu.async_copy(rows, acc_sm.at[di], ssem, add=True)

    meta_issue(0, bufs[0], drain=False)
    meta_issue(1, bufs[1], drain=False)
    gather_issue(0, bufs[0])

    def chunk_body(c, _):
        n2 = c + 2
        for k in range(NB):
            @pl.when((n2 < NCH) & (n2 % NB == k) & (c >= 2))
            def _(k=k, n2=n2):
                meta_issue(n2, bufs[k], drain=True)

            @pl.when((n2 < NCH) & (n2 % NB == k) & (c < 2))
            def _(k=k, n2=n2):
                meta_issue(n2, bufs[k], drain=False)

        n1 = c + 1
        for k in range(NB):
            @pl.when((n1 < NCH) & (n1 % NB == k))
            def _(k=k, n1=n1):
                gather_issue(n1, bufs[k])

        for k in range(NB):
            @pl.when(c % NB == k)
            def _(k=k):
                consume(bufs[k])
        return 0
    lax.fori_loop(0, NCH, chunk_body, 0)

    for k in range(NB):
        gi, di, sv, rows, msem, gsem, ssem = bufs[k]
        pltpu.make_async_copy(rows, acc_sm.at[di], ssem).wait()

    plsc.subcore_barrier()
    ob = pl.multiple_of(sid * 624, 8)

    @pl.when(sid < NS - 1)
    def _():
        pltpu.sync_copy(acc_sm.at[pl.ds(ob, 624)],
                        out_hbm.at[cid, pl.ds(ob, 624)])

    @pl.when(sid == NS - 1)
    def _():
        pltpu.sync_copy(acc_sm.at[pl.ds(ob, 640)],
                        out_hbm.at[cid, pl.ds(ob, 640)])


def _sc_scatter(y, gi, dst, s, D):
    ring_bufs = [
        pltpu.VMEM((CHUNK,), jnp.int32),
        pltpu.VMEM((CHUNK,), jnp.int32),
        pltpu.VMEM((CHUNK,), jnp.float32),
        pltpu.VMEM((CHUNK, D), jnp.float32),
    ] * 4
    sems = [pltpu.SemaphoreType.DMA] * 12
    return pl.kernel(
        functools.partial(_scatter_body, D),
        out_type=jax.ShapeDtypeStruct((NC, N_NODES, D), jnp.float32),
        mesh=_mesh,
        compiler_params=_sc_params,
        scratch_types=ring_bufs + sems + [
            pltpu.VMEM_SHARED((N_NODES, D), jnp.float32),
        ],
    )(y, gi, dst, s)


def _tc1_body(x_ref, w_ref, root_ref, b_ref, y_ref, h0_ref):
    xb = x_ref[...]
    for r in range(N_REL):
        y_ref[:, r, :] = jnp.dot(xb, w_ref[r], preferred_element_type=jnp.float32)
    h0_ref[...] = (jnp.dot(xb, root_ref[...], preferred_element_type=jnp.float32)
                   + b_ref[...])


def _tc_transform1(x, W, root, b, d_in, d_out):
    blk = 400
    grid = (N_NODES // blk,)
    return pl.pallas_call(
        _tc1_body,
        grid=grid,
        in_specs=[
            pl.BlockSpec((blk, d_in), lambda i: (i, 0)),
            pl.BlockSpec((N_REL, d_in, d_out), lambda i: (0, 0, 0)),
            pl.BlockSpec((d_in, d_out), lambda i: (0, 0)),
            pl.BlockSpec((1, d_out), lambda i: (0, 0)),
        ],
        out_specs=[
            pl.BlockSpec((blk, N_REL, d_out), lambda i: (i, 0, 0)),
            pl.BlockSpec((blk, d_out), lambda i: (i, 0)),
        ],
        out_shape=[
            jax.ShapeDtypeStruct((N_NODES, N_REL, d_out), jnp.float32),
            jax.ShapeDtypeStruct((N_NODES, d_out), jnp.float32),
        ],
        compiler_params=pltpu.CompilerParams(
            dimension_semantics=("parallel",)),
    )(x, W, root, b)


def _tc2_body(h0_ref, pa_ref, pb_ref, w_ref, root_ref, b_ref, y_ref, g0_ref):
    h = jnp.maximum(h0_ref[...] + pa_ref[...] + pb_ref[...], 0.0)
    for r in range(N_REL):
        y_ref[:, r, :] = jnp.dot(h, w_ref[r], preferred_element_type=jnp.float32)
    g0_ref[...] = (jnp.dot(h, root_ref[...], preferred_element_type=jnp.float32)
                   + b_ref[...])


def _tc_transform2(h0, pa, pb, W, root, b, d_in, d_w, d_out):
    blk = 400
    grid = (N_NODES // blk,)
    return pl.pallas_call(
        _tc2_body,
        grid=grid,
        in_specs=[
            pl.BlockSpec((blk, d_in), lambda i: (i, 0)),
            pl.BlockSpec((blk, d_in), lambda i: (i, 0)),
            pl.BlockSpec((blk, d_in), lambda i: (i, 0)),
            pl.BlockSpec((N_REL, d_in, d_w), lambda i: (0, 0, 0)),
            pl.BlockSpec((d_in, d_out), lambda i: (0, 0)),
            pl.BlockSpec((1, d_out), lambda i: (0, 0)),
        ],
        out_specs=[
            pl.BlockSpec((blk, N_REL, d_w), lambda i: (i, 0, 0)),
            pl.BlockSpec((blk, d_out), lambda i: (i, 0)),
        ],
        out_shape=[
            jax.ShapeDtypeStruct((N_NODES, N_REL, d_w), jnp.float32),
            jax.ShapeDtypeStruct((N_NODES, d_out), jnp.float32),
        ],
        compiler_params=pltpu.CompilerParams(
            dimension_semantics=("parallel",)),
    )(h0, pa, pb, W, root, b)


def _fin_body(g0_ref, pa_ref, pb_ref, o_ref, acc_ref):
    i = pl.program_id(0)

    @pl.when(i == 0)
    def _():
        acc_ref[...] = jnp.zeros_like(acc_ref)

    hb = g0_ref[...] + pa_ref[...] + pb_ref[...]
    acc_ref[...] += jnp.sum(hb, axis=0, keepdims=True)

    @pl.when(i == pl.num_programs(0) - 1)
    def _():
        g = acc_ref[...] / N_NODES
        m = jnp.max(g)
        lse = jnp.log(jnp.sum(jnp.exp(g - m))) + m
        o_ref[...] = g - lse


def _tc_finish(g0, pa, pb, d):
    blk = 400
    grid = (N_NODES // blk,)
    return pl.pallas_call(
        _fin_body,
        grid=grid,
        in_specs=[
            pl.BlockSpec((blk, d), lambda i: (i, 0)),
            pl.BlockSpec((blk, d), lambda i: (i, 0)),
            pl.BlockSpec((blk, d), lambda i: (i, 0)),
        ],
        out_specs=pl.BlockSpec((1, d), lambda i: (0, 0)),
        out_shape=jax.ShapeDtypeStruct((1, d), jnp.float32),
        scratch_shapes=[pltpu.VMEM((1, d), jnp.float32)],
        compiler_params=pltpu.CompilerParams(
            dimension_semantics=("arbitrary",)),
    )(g0, pa, pb)


def kernel(x, edge_index, edge_type, W1, root1, b1, W2, root2, b2):
    src = edge_index[0]
    dst = edge_index[1]
    et = edge_type

    s, gi = _sc_scales(et, dst, src)

    y1, h0 = _tc_transform1(x, W1, root1, b1.reshape(1, D_HID), D_IN, D_HID)
    p1 = _sc_scatter(y1.reshape(N_NODES * N_REL, D_HID), gi, dst, s, D_HID)

    W2p = jnp.pad(W2, ((0, 0), (0, 0), (0, D_HID - D_OUT)))
    y2, g0 = _tc_transform2(h0, p1[0], p1[1], W2p, root2, b2.reshape(1, D_OUT),
                            D_HID, D_HID, D_OUT)
    p2 = _sc_scatter(y2.reshape(N_NODES * N_REL, D_HID), gi, dst, s, D_HID)

    return _tc_finish(g0, p2[0, :, :D_OUT], p2[1, :, :D_OUT], D_OUT)


# speedup vs baseline: 43.4143x; 1.2056x over previous
import functools

import jax
import jax.numpy as jnp
from jax import lax
from jax.experimental import pallas as pl
from jax.experimental.pallas import tpu as pltpu
from jax.experimental.pallas import tpu_sc as plsc

N_NODES = 10000
N_EDGES = 320000
D_IN = 128
D_HID = 128
D_OUT = 64
N_REL = 8

NC, NS = 2, 16
NW = NC * NS
E_TILE = N_EDGES // NW
E_CNT = N_EDGES // NS
KEYS = N_NODES * N_REL
KEYS_PAD = 81920
RED = KEYS_PAD // NS
CHUNK = 80
BLK = 2000
ROWS_TILE = N_NODES // NS

_mesh = plsc.VectorSubcoreMesh(core_axis_name="c", subcore_axis_name="s")
_sc_params = pltpu.CompilerParams(needs_layout_passes=False)

_Z16 = functools.partial(jnp.zeros, (16,), jnp.float32)
_O16 = functools.partial(jnp.ones, (16,), jnp.float32)


HROWS = KEYS_PAD // 128


def _scales_body(et_hbm, dst_hbm, src_hbm, s_hbm, gi_hbm,
                 hist2, tbuf, dbuf, sbuf, srcb, gibuf, zbuf, ridx, acc_sm):
    cid = lax.axis_index("c")
    sid = lax.axis_index("s")

    def zh(i, _):
        for q in range(8):
            hist2[i, pl.ds(q * 16, 16)] = _Z16()
        return 0
    lax.fori_loop(0, HROWS, zh, 0)
    def zz(i, _):
        for q in range(8):
            zbuf[i, pl.ds(q * 16, 16)] = _Z16()
        return 0
    lax.fori_loop(0, HROWS // NS, zz, 0)
    pltpu.sync_copy(zbuf, acc_sm.at[pl.ds(sid * (HROWS // NS), HROWS // NS)])
    plsc.subcore_barrier()

    for blk in range(E_CNT // BLK):
        base = sid * E_CNT + blk * BLK
        pltpu.sync_copy(et_hbm.at[pl.ds(base, BLK)], tbuf)
        pltpu.sync_copy(dst_hbm.at[pl.ds(base, BLK)], dbuf)

        def cbody(i, _):
            t = tbuf[pl.ds(i * 16, 16)]
            d = dbuf[pl.ds(i * 16, 16)]
            key = d * N_REL + t
            plsc.addupdate_scatter(
                hist2, [key >> 7, key & 127], _O16())
            return 0
        lax.fori_loop(0, BLK // 16, cbody, 0)

    for k in range(HROWS // 128):
        for j in range(8):
            ridx[pl.ds(j * 16, 16)] = (
                lax.iota(jnp.int32, 16) + (k * 128 + j * 16))
        pltpu.sync_copy(hist2.at[pl.ds(k * 128, 128)],
                        acc_sm.at[ridx], add=True)
    plsc.subcore_barrier()

    pltpu.sync_copy(acc_sm, hist2)
    ebase = (cid * NS + sid) * E_TILE
    for blk in range(E_TILE // BLK):
        base = ebase + blk * BLK
        pltpu.sync_copy(et_hbm.at[pl.ds(base, BLK)], tbuf)
        pltpu.sync_copy(dst_hbm.at[pl.ds(base, BLK)], dbuf)
        pltpu.sync_copy(src_hbm.at[pl.ds(base, BLK)], srcb)

        @plsc.parallel_loop(0, BLK // 16, 1, unroll=4)
        def _(i):
            t = tbuf[pl.ds(i * 16, 16)]
            d = dbuf[pl.ds(i * 16, 16)]
            sr = srcb[pl.ds(i * 16, 16)]
            key = d * N_REL + t
            c = plsc.load_gather(hist2, [key >> 7, key & 127])
            sbuf[pl.ds(i * 16, 16)] = 1.0 / jnp.maximum(c, 1.0)
            gibuf[pl.ds(i * 16, 16)] = sr * N_REL + t
        pltpu.sync_copy(sbuf, s_hbm.at[pl.ds(base, BLK)])
        pltpu.sync_copy(gibuf, gi_hbm.at[pl.ds(base, BLK)])


def _sc_scales(et, dst, src):
    return pl.kernel(
        _scales_body,
        out_type=[jax.ShapeDtypeStruct((N_EDGES,), jnp.float32),
                  jax.ShapeDtypeStruct((N_EDGES,), jnp.int32)],
        mesh=_mesh,
        compiler_params=_sc_params,
        scratch_types=[
            pltpu.VMEM((HROWS, 128), jnp.float32),
            pltpu.VMEM((BLK,), jnp.int32),
            pltpu.VMEM((BLK,), jnp.int32),
            pltpu.VMEM((BLK,), jnp.float32),
            pltpu.VMEM((BLK,), jnp.int32),
            pltpu.VMEM((BLK,), jnp.int32),
            pltpu.VMEM((HROWS // NS, 128), jnp.float32),
            pltpu.VMEM((128,), jnp.int32),
            pltpu.VMEM_SHARED((HROWS, 128), jnp.float32),
        ],
    )(et, dst, src)


def _scatter_body(D, y_hbm, gi_hbm, di_hbm, s_hbm, out_hbm,
                  gi1, di1, s1, rows1, gi2, di2, s2, rows2,
                  gi3, di3, s3, rows3, gi4, di4, s4, rows4,
                  msem1, gsem1, ssem1, msem2, gsem2, ssem2,
                  msem3, gsem3, ssem3, msem4, gsem4, ssem4, acc_sm):
    cid = lax.axis_index("c")
    sid = lax.axis_index("s")
    wid = cid * NS + sid
    nq = D // 16

    def zb(i, _):
        for q in range(nq):
            rows1[i, pl.ds(q * 16, 16)] = _Z16()
        return 0
    lax.fori_loop(0, CHUNK, zb, 0)
    rbase = sid * ROWS_TILE
    for q in range(ROWS_TILE // CHUNK):
        pltpu.sync_copy(rows1, acc_sm.at[pl.ds(rbase + q * CHUNK, CHUNK)])
    rem = ROWS_TILE % CHUNK
    if rem:
        pltpu.sync_copy(rows1.at[pl.ds(0, rem)],
                        acc_sm.at[pl.ds(rbase + ROWS_TILE - rem, rem)])
    plsc.subcore_barrier()

    ebase = wid * E_TILE
    bufs = ((gi1, di1, s1, rows1, msem1, gsem1, ssem1),
            (gi2, di2, s2, rows2, msem2, gsem2, ssem2),
            (gi3, di3, s3, rows3, msem3, gsem3, ssem3),
            (gi4, di4, s4, rows4, msem4, gsem4, ssem4))
    NB = 4
    NCH = E_TILE // CHUNK

    def meta_issue(c, buf, drain):
        gi, di, sv, rows, msem, gsem, ssem = buf
        if drain:
            pltpu.make_async_copy(rows, acc_sm.at[di], ssem).wait()
        base = ebase + c * CHUNK
        pltpu.async_copy(gi_hbm.at[pl.ds(base, CHUNK)], gi, msem)
        pltpu.async_copy(di_hbm.at[pl.ds(base, CHUNK)], di, msem)
        pltpu.async_copy(s_hbm.at[pl.ds(base, CHUNK)], sv, msem)

    def gather_issue(c, buf):
        gi, di, sv, rows, msem, gsem, ssem = buf
        base = ebase + c * CHUNK
        pltpu.make_async_copy(gi_hbm.at[pl.ds(base, CHUNK)], gi, msem).wait()
        pltpu.make_async_copy(di_hbm.at[pl.ds(base, CHUNK)], di, msem).wait()
        pltpu.make_async_copy(s_hbm.at[pl.ds(base, CHUNK)], sv, msem).wait()
        pltpu.async_copy(y_hbm.at[gi], rows, gsem)

    def consume(buf):
        gi, di, sv, rows, msem, gsem, ssem = buf
        pltpu.make_async_copy(y_hbm.at[gi], rows, gsem).wait()

        @plsc.parallel_loop(0, CHUNK, 1, unroll=4)
        def _(j):
            sj = plsc.load_gather(sv, [jnp.full((16,), j, jnp.int32)])
            for q in range(nq):
                rows[j, pl.ds(q * 16, 16)] = rows[j, pl.ds(q * 16, 16)] * sj
        pltpu.async_copy(rows, acc_sm.at[di], ssem, add=True)

    meta_issue(0, bufs[0], drain=False)
    meta_issue(1, bufs[1], drain=False)
    gather_issue(0, bufs[0])

    def chunk_body(c, _):
        n2 = c + 2
        for k in range(NB):
            @pl.when((n2 < NCH) & (n2 % NB == k) & (c >= 2))
            def _(k=k, n2=n2):
                meta_issue(n2, bufs[k], drain=True)

            @pl.when((n2 < NCH) & (n2 % NB == k) & (c < 2))
            def _(k=k, n2=n2):
                meta_issue(n2, bufs[k], drain=False)

        n1 = c + 1
        for k in range(NB):
            @pl.when((n1 < NCH) & (n1 % NB == k))
            def _(k=k, n1=n1):
                gather_issue(n1, bufs[k])

        for k in range(NB):
            @pl.when(c % NB == k)
            def _(k=k):
                consume(bufs[k])
        return 0
    lax.fori_loop(0, NCH, chunk_body, 0)

    for k in range(NB):
        gi, di, sv, rows, msem, gsem, ssem = bufs[k]
        pltpu.make_async_copy(rows, acc_sm.at[di], ssem).wait()

    plsc.subcore_barrier()
    ob = pl.multiple_of(sid * 624, 8)

    @pl.when(sid < NS - 1)
    def _():
        pltpu.sync_copy(acc_sm.at[pl.ds(ob, 624)],
                        out_hbm.at[cid, pl.ds(ob, 624)])

    @pl.when(sid == NS - 1)
    def _():
        pltpu.sync_copy(acc_sm.at[pl.ds(ob, 640)],
                        out_hbm.at[cid, pl.ds(ob, 640)])


def _sc_scatter(y, gi, dst, s, D):
    ring_bufs = [
        pltpu.VMEM((CHUNK,), jnp.int32),
        pltpu.VMEM((CHUNK,), jnp.int32),
        pltpu.VMEM((CHUNK,), jnp.float32),
        pltpu.VMEM((CHUNK, D), jnp.float32),
    ] * 4
    sems = [pltpu.SemaphoreType.DMA] * 12
    return pl.kernel(
        functools.partial(_scatter_body, D),
        out_type=jax.ShapeDtypeStruct((NC, N_NODES, D), jnp.float32),
        mesh=_mesh,
        compiler_params=_sc_params,
        scratch_types=ring_bufs + sems + [
            pltpu.VMEM_SHARED((N_NODES, D), jnp.float32),
        ],
    )(y, gi, dst, s)


def _reduce_body(y_hbm, gi_hbm, s_hbm, out_hbm,
                 gi1, s1, rows1, gi2, s2, rows2, gi3, s3, rows3,
                 gi4, s4, rows4, gi5, s5, rows5, gi6, s6, rows6,
                 msem1, gsem1, msem2, gsem2, msem3, gsem3,
                 msem4, gsem4, msem5, gsem5, msem6, gsem6, acc8):
    cid = lax.axis_index("c")
    sid = lax.axis_index("s")
    wid = cid * NS + sid

    def za(i, _):
        for q in range(8):
            acc8[i, pl.ds(q * 16, 16)] = _Z16()
        return 0
    lax.fori_loop(0, 8, za, 0)

    ebase = wid * E_TILE
    bufs = ((gi1, s1, rows1, msem1, gsem1), (gi2, s2, rows2, msem2, gsem2),
            (gi3, s3, rows3, msem3, gsem3), (gi4, s4, rows4, msem4, gsem4),
            (gi5, s5, rows5, msem5, gsem5), (gi6, s6, rows6, msem6, gsem6))
    NB = 6
    NCH = E_TILE // CHUNK

    def meta_issue(c, buf):
        gi, sv, rows, msem, gsem = buf
        base = ebase + c * CHUNK
        pltpu.async_copy(gi_hbm.at[pl.ds(base, CHUNK)], gi, msem)
        pltpu.async_copy(s_hbm.at[pl.ds(base, CHUNK)], sv, msem)

    def gather_issue(c, buf):
        gi, sv, rows, msem, gsem = buf
        base = ebase + c * CHUNK
        pltpu.make_async_copy(gi_hbm.at[pl.ds(base, CHUNK)], gi, msem).wait()
        pltpu.make_async_copy(s_hbm.at[pl.ds(base, CHUNK)], sv, msem).wait()
        pltpu.async_copy(y_hbm.at[gi], rows, gsem)

    def consume(buf):
        gi, sv, rows, msem, gsem = buf
        pltpu.make_async_copy(y_hbm.at[gi], rows, gsem).wait()

        def rbody(j, acc):
            sj = plsc.load_gather(sv, [jnp.full((16,), j, jnp.int32)])
            return tuple(acc[q] + rows[j, pl.ds(q * 16, 16)] * sj
                         for q in range(8))
        acc = lax.fori_loop(0, CHUNK, rbody, tuple(_Z16() for _ in range(8)))
        for q in range(8):
            acc8[0, pl.ds(q * 16, 16)] = acc8[0, pl.ds(q * 16, 16)] + acc[q]

    for n in range(4):
        meta_issue(n, bufs[n])
    gather_issue(0, bufs[0])
    gather_issue(1, bufs[1])

    def chunk_body(c, _):
        n4 = c + 4
        for k in range(NB):
            @pl.when((n4 < NCH) & (n4 % NB == k))
            def _(k=k, n4=n4):
                meta_issue(n4, bufs[k])

        n2 = c + 2
        for k in range(NB):
            @pl.when((n2 < NCH) & (n2 % NB == k))
            def _(k=k, n2=n2):
                gather_issue(n2, bufs[k])

        for k in range(NB):
            @pl.when(c % NB == k)
            def _(k=k):
                consume(bufs[k])
        return 0
    lax.fori_loop(0, NCH, chunk_body, 0)

    pltpu.sync_copy(acc8, out_hbm.at[cid, sid])


def _sc_reduce(y, gi, s):
    ring = [
        pltpu.VMEM((CHUNK,), jnp.int32),
        pltpu.VMEM((CHUNK,), jnp.float32),
        pltpu.VMEM((CHUNK, D_HID), jnp.float32),
    ] * 6
    sems = [pltpu.SemaphoreType.DMA] * 12
    return pl.kernel(
        _reduce_body,
        out_type=jax.ShapeDtypeStruct((NC, NS, 8, D_HID), jnp.float32),
        mesh=_mesh,
        compiler_params=_sc_params,
        scratch_types=ring + sems + [pltpu.VMEM((8, D_HID), jnp.float32)],
    )(y, gi, s)


def _tc1_body(x_ref, w_ref, root_ref, b_ref, y_ref, h0_ref):
    xb = x_ref[...]
    for r in range(N_REL):
        y_ref[:, r, :] = jnp.dot(xb, w_ref[r], preferred_element_type=jnp.float32)
    h0_ref[...] = (jnp.dot(xb, root_ref[...], preferred_element_type=jnp.float32)
                   + b_ref[...])


def _tc_transform1(x, W, root, b, d_in, d_out):
    blk = 400
    grid = (N_NODES // blk,)
    return pl.pallas_call(
        _tc1_body,
        grid=grid,
        in_specs=[
            pl.BlockSpec((blk, d_in), lambda i: (i, 0)),
            pl.BlockSpec((N_REL, d_in, d_out), lambda i: (0, 0, 0)),
            pl.BlockSpec((d_in, d_out), lambda i: (0, 0)),
            pl.BlockSpec((1, d_out), lambda i: (0, 0)),
        ],
        out_specs=[
            pl.BlockSpec((blk, N_REL, d_out), lambda i: (i, 0, 0)),
            pl.BlockSpec((blk, d_out), lambda i: (i, 0)),
        ],
        out_shape=[
            jax.ShapeDtypeStruct((N_NODES, N_REL, d_out), jnp.float32),
            jax.ShapeDtypeStruct((N_NODES, d_out), jnp.float32),
        ],
        compiler_params=pltpu.CompilerParams(
            dimension_semantics=("parallel",)),
    )(x, W, root, b)


def _tc2_body(h0_ref, pa_ref, pb_ref, w_ref, y_ref, hs_ref):
    h = jnp.maximum(h0_ref[...] + pa_ref[...] + pb_ref[...], 0.0)
    for r in range(N_REL):
        y_ref[:, r, :] = jnp.dot(h, w_ref[r], preferred_element_type=jnp.float32)
    hs_ref[...] = jnp.sum(h, axis=0, keepdims=True)[None]


def _tc_transform2(h0, pa, pb, W, d_in, d_w):
    blk = 400
    grid = (N_NODES // blk,)
    return pl.pallas_call(
        _tc2_body,
        grid=grid,
        in_specs=[
            pl.BlockSpec((blk, d_in), lambda i: (i, 0)),
            pl.BlockSpec((blk, d_in), lambda i: (i, 0)),
            pl.BlockSpec((blk, d_in), lambda i: (i, 0)),
            pl.BlockSpec((N_REL, d_in, d_w), lambda i: (0, 0, 0)),
        ],
        out_specs=[
            pl.BlockSpec((blk, N_REL, d_w), lambda i: (i, 0, 0)),
            pl.BlockSpec((1, 1, d_in), lambda i: (i, 0, 0)),
        ],
        out_shape=[
            jax.ShapeDtypeStruct((N_NODES, N_REL, d_w), jnp.float32),
            jax.ShapeDtypeStruct((N_NODES // blk, 1, d_in), jnp.float32),
        ],
        compiler_params=pltpu.CompilerParams(
            dimension_semantics=("parallel",)),
    )(h0, pa, pb, W)


def _fin_body(hs_ref, root_ref, b_ref, ep_ref, o_ref):
    hv = jnp.sum(hs_ref[...], axis=(0, 1))[None, :]
    ep = jnp.sum(ep_ref[...], axis=(0, 1, 2))[None, :D_OUT]
    g = (jnp.dot(hv, root_ref[...], preferred_element_type=jnp.float32)
         + ep) / N_NODES + b_ref[...]
    m = jnp.max(g)
    lse = jnp.log(jnp.sum(jnp.exp(g - m))) + m
    o_ref[...] = g - lse


def _tc_finish(hsum, root, b, ep):
    nb = hsum.shape[0]
    return pl.pallas_call(
        _fin_body,
        in_specs=[
            pl.BlockSpec((nb, 1, D_HID), lambda: (0, 0, 0)),
            pl.BlockSpec((D_HID, D_OUT), lambda: (0, 0)),
            pl.BlockSpec((1, D_OUT), lambda: (0, 0)),
            pl.BlockSpec((NC, NS, 8, D_HID), lambda: (0, 0, 0, 0)),
        ],
        out_specs=pl.BlockSpec((1, D_OUT), lambda: (0, 0)),
        out_shape=jax.ShapeDtypeStruct((1, D_OUT), jnp.float32),
    )(hsum, root, b, ep)


def kernel(x, edge_index, edge_type, W1, root1, b1, W2, root2, b2):
    src = edge_index[0]
    dst = edge_index[1]
    et = edge_type

    s, gi = _sc_scales(et, dst, src)

    y1, h0 = _tc_transform1(x, W1, root1, b1.reshape(1, D_HID), D_IN, D_HID)
    p1 = _sc_scatter(y1.reshape(N_NODES * N_REL, D_HID), gi, dst, s, D_HID)

    W2p = jnp.pad(W2, ((0, 0), (0, 0), (0, D_HID - D_OUT)))
    y2, hsum = _tc_transform2(h0, p1[0], p1[1], W2p, D_HID, D_HID)
    ep = _sc_reduce(y2.reshape(N_NODES * N_REL, D_HID), gi, s)

    return _tc_finish(hsum, root2, b2.reshape(1, D_OUT), ep)


# speedup vs baseline: 46.3460x; 1.0675x over previous
import functools

import jax
import jax.numpy as jnp
from jax import lax
from jax.experimental import pallas as pl
from jax.experimental.pallas import tpu as pltpu
from jax.experimental.pallas import tpu_sc as plsc

N_NODES = 10000
N_EDGES = 320000
D_IN = 128
D_HID = 128
D_OUT = 64
N_REL = 8

NC, NS = 2, 16
NW = NC * NS
E_TILE = N_EDGES // NW
E_CNT = N_EDGES // NS
KEYS = N_NODES * N_REL
KEYS_PAD = 81920
RED = KEYS_PAD // NS
CHUNK = 80
BLK = 2000
ROWS_TILE = N_NODES // NS

_mesh = plsc.VectorSubcoreMesh(core_axis_name="c", subcore_axis_name="s")
_sc_params = pltpu.CompilerParams(needs_layout_passes=False)

_Z16 = functools.partial(jnp.zeros, (16,), jnp.float32)
_O16 = functools.partial(jnp.ones, (16,), jnp.float32)


HROWS = KEYS_PAD // 128


def _scales_body(et_hbm, dst_hbm, src_hbm, s_hbm, gi_hbm,
                 hist2, tb0, db0, sr0, tb1, db1, sr1, sb0, gb0, sb1, gb1,
                 zbuf, ridx, isem0, isem1, osem0, osem1, acc_sm):
    cid = lax.axis_index("c")
    sid = lax.axis_index("s")
    ins = ((tb0, db0, sr0, isem0, sb0, gb0, osem0),
           (tb1, db1, sr1, isem1, sb1, gb1, osem1))

    def stage_a(blk, st):
        tb, db, sr, isem = st[:4]
        base = sid * E_CNT + blk * BLK
        pltpu.async_copy(et_hbm.at[pl.ds(base, BLK)], tb, isem)
        pltpu.async_copy(dst_hbm.at[pl.ds(base, BLK)], db, isem)

    def wait_a(blk, st):
        tb, db, sr, isem = st[:4]
        base = sid * E_CNT + blk * BLK
        pltpu.make_async_copy(et_hbm.at[pl.ds(base, BLK)], tb, isem).wait()
        pltpu.make_async_copy(dst_hbm.at[pl.ds(base, BLK)], db, isem).wait()

    stage_a(0, ins[0])
    stage_a(1, ins[1])

    def zh(i, _):
        for q in range(8):
            hist2[i, pl.ds(q * 16, 16)] = _Z16()
        return 0
    lax.fori_loop(0, HROWS, zh, 0)
    def zz(i, _):
        for q in range(8):
            zbuf[i, pl.ds(q * 16, 16)] = _Z16()
        return 0
    lax.fori_loop(0, HROWS // NS, zz, 0)
    pltpu.sync_copy(zbuf, acc_sm.at[pl.ds(sid * (HROWS // NS), HROWS // NS)])
    plsc.subcore_barrier()

    NA = E_CNT // BLK
    for blk in range(NA):
        st = ins[blk % 2]
        wait_a(blk, st)

        def cbody(i, _):
            t = st[0][pl.ds(i * 16, 16)]
            d = st[1][pl.ds(i * 16, 16)]
            key = d * N_REL + t
            plsc.addupdate_scatter(
                hist2, [key >> 7, key & 127], _O16())
            return 0
        lax.fori_loop(0, BLK // 16, cbody, 0)
        if blk + 2 < NA:
            stage_a(blk + 2, st)

    for k in range(HROWS // 128):
        for j in range(8):
            ridx[pl.ds(j * 16, 16)] = (
                lax.iota(jnp.int32, 16) + (k * 128 + j * 16))
        pltpu.sync_copy(hist2.at[pl.ds(k * 128, 128)],
                        acc_sm.at[ridx], add=True)
    plsc.subcore_barrier()

    ebase = (cid * NS + sid) * E_TILE

    def stage_c(blk, st):
        tb, db, sr, isem = st[:4]
        base = ebase + blk * BLK
        pltpu.async_copy(et_hbm.at[pl.ds(base, BLK)], tb, isem)
        pltpu.async_copy(dst_hbm.at[pl.ds(base, BLK)], db, isem)
        pltpu.async_copy(src_hbm.at[pl.ds(base, BLK)], sr, isem)

    def wait_c(blk, st):
        tb, db, sr, isem = st[:4]
        base = ebase + blk * BLK
        pltpu.make_async_copy(et_hbm.at[pl.ds(base, BLK)], tb, isem).wait()
        pltpu.make_async_copy(dst_hbm.at[pl.ds(base, BLK)], db, isem).wait()
        pltpu.make_async_copy(src_hbm.at[pl.ds(base, BLK)], sr, isem).wait()

    def wait_out(blk, st):
        sb, gb, osem = st[4:]
        base = ebase + blk * BLK
        pltpu.make_async_copy(sb, s_hbm.at[pl.ds(base, BLK)], osem).wait()
        pltpu.make_async_copy(gb, gi_hbm.at[pl.ds(base, BLK)], osem).wait()

    pltpu.sync_copy(acc_sm, hist2)
    NCB = E_TILE // BLK
    stage_c(0, ins[0])
    stage_c(1, ins[1])
    for blk in range(NCB):
        st = ins[blk % 2]
        tb, db, sr, isem, sb, gb, osem = st
        wait_c(blk, st)
        if blk >= 2:
            wait_out(blk - 2, st)

        @plsc.parallel_loop(0, BLK // 16, 1, unroll=4)
        def _(i):
            t = tb[pl.ds(i * 16, 16)]
            d = db[pl.ds(i * 16, 16)]
            s_ = sr[pl.ds(i * 16, 16)]
            key = d * N_REL + t
            c = plsc.load_gather(hist2, [key >> 7, key & 127])
            sb[pl.ds(i * 16, 16)] = 1.0 / jnp.maximum(c, 1.0)
            gb[pl.ds(i * 16, 16)] = s_ * N_REL + t
        base = ebase + blk * BLK
        pltpu.async_copy(sb, s_hbm.at[pl.ds(base, BLK)], osem)
        pltpu.async_copy(gb, gi_hbm.at[pl.ds(base, BLK)], osem)
        if blk + 2 < NCB:
            stage_c(blk + 2, st)
    wait_out(NCB - 2, ins[(NCB - 2) % 2])
    wait_out(NCB - 1, ins[(NCB - 1) % 2])


def _sc_scales(et, dst, src):
    return pl.kernel(
        _scales_body,
        out_type=[jax.ShapeDtypeStruct((N_EDGES,), jnp.float32),
                  jax.ShapeDtypeStruct((N_EDGES,), jnp.int32)],
        mesh=_mesh,
        compiler_params=_sc_params,
        scratch_types=[
            pltpu.VMEM((HROWS, 128), jnp.float32),
            pltpu.VMEM((BLK,), jnp.int32),
            pltpu.VMEM((BLK,), jnp.int32),
            pltpu.VMEM((BLK,), jnp.int32),
            pltpu.VMEM((BLK,), jnp.int32),
            pltpu.VMEM((BLK,), jnp.int32),
            pltpu.VMEM((BLK,), jnp.int32),
            pltpu.VMEM((BLK,), jnp.float32),
            pltpu.VMEM((BLK,), jnp.int32),
            pltpu.VMEM((BLK,), jnp.float32),
            pltpu.VMEM((BLK,), jnp.int32),
            pltpu.VMEM((HROWS // NS, 128), jnp.float32),
            pltpu.VMEM((128,), jnp.int32),
            pltpu.SemaphoreType.DMA,
            pltpu.SemaphoreType.DMA,
            pltpu.SemaphoreType.DMA,
            pltpu.SemaphoreType.DMA,
            pltpu.VMEM_SHARED((HROWS, 128), jnp.float32),
        ],
    )(et, dst, src)


def _scatter_body(D, y_hbm, gi_hbm, di_hbm, s_hbm, out_hbm,
                  gi1, di1, s1, rows1, gi2, di2, s2, rows2,
                  gi3, di3, s3, rows3, gi4, di4, s4, rows4,
                  msem1, gsem1, ssem1, msem2, gsem2, ssem2,
                  msem3, gsem3, ssem3, msem4, gsem4, ssem4, acc_sm):
    cid = lax.axis_index("c")
    sid = lax.axis_index("s")
    wid = cid * NS + sid
    nq = D // 16

    def zb(i, _):
        for q in range(nq):
            rows1[i, pl.ds(q * 16, 16)] = _Z16()
        return 0
    lax.fori_loop(0, CHUNK, zb, 0)
    rbase = sid * ROWS_TILE
    for q in range(ROWS_TILE // CHUNK):
        pltpu.sync_copy(rows1, acc_sm.at[pl.ds(rbase + q * CHUNK, CHUNK)])
    rem = ROWS_TILE % CHUNK
    if rem:
        pltpu.sync_copy(rows1.at[pl.ds(0, rem)],
                        acc_sm.at[pl.ds(rbase + ROWS_TILE - rem, rem)])
    plsc.subcore_barrier()

    ebase = wid * E_TILE
    bufs = ((gi1, di1, s1, rows1, msem1, gsem1, ssem1),
            (gi2, di2, s2, rows2, msem2, gsem2, ssem2),
            (gi3, di3, s3, rows3, msem3, gsem3, ssem3),
            (gi4, di4, s4, rows4, msem4, gsem4, ssem4))
    NB = 4
    NCH = E_TILE // CHUNK

    def meta_issue(c, buf, drain):
        gi, di, sv, rows, msem, gsem, ssem = buf
        if drain:
            pltpu.make_async_copy(rows, acc_sm.at[di], ssem).wait()
        base = ebase + c * CHUNK
        pltpu.async_copy(gi_hbm.at[pl.ds(base, CHUNK)], gi, msem)
        pltpu.async_copy(di_hbm.at[pl.ds(base, CHUNK)], di, msem)
        pltpu.async_copy(s_hbm.at[pl.ds(base, CHUNK)], sv, msem)

    def gather_issue(c, buf):
        gi, di, sv, rows, msem, gsem, ssem = buf
        base = ebase + c * CHUNK
        pltpu.make_async_copy(gi_hbm.at[pl.ds(base, CHUNK)], gi, msem).wait()
        pltpu.make_async_copy(di_hbm.at[pl.ds(base, CHUNK)], di, msem).wait()
        pltpu.make_async_copy(s_hbm.at[pl.ds(base, CHUNK)], sv, msem).wait()
        pltpu.async_copy(y_hbm.at[gi], rows, gsem)

    def consume(buf):
        gi, di, sv, rows, msem, gsem, ssem = buf
        pltpu.make_async_copy(y_hbm.at[gi], rows, gsem).wait()

        @plsc.parallel_loop(0, CHUNK, 1, unroll=4)
        def _(j):
            sj = plsc.load_gather(sv, [jnp.full((16,), j, jnp.int32)])
            for q in range(nq):
                rows[j, pl.ds(q * 16, 16)] = rows[j, pl.ds(q * 16, 16)] * sj
        pltpu.async_copy(rows, acc_sm.at[di], ssem, add=True)

    meta_issue(0, bufs[0], drain=False)
    meta_issue(1, bufs[1], drain=False)
    gather_issue(0, bufs[0])

    def chunk_body(c, _):
        n2 = c + 2
        for k in range(NB):
            @pl.when((n2 < NCH) & (n2 % NB == k) & (c >= 2))
            def _(k=k, n2=n2):
                meta_issue(n2, bufs[k], drain=True)

            @pl.when((n2 < NCH) & (n2 % NB == k) & (c < 2))
            def _(k=k, n2=n2):
                meta_issue(n2, bufs[k], drain=False)

        n1 = c + 1
        for k in range(NB):
            @pl.when((n1 < NCH) & (n1 % NB == k))
            def _(k=k, n1=n1):
                gather_issue(n1, bufs[k])

        for k in range(NB):
            @pl.when(c % NB == k)
            def _(k=k):
                consume(bufs[k])
        return 0
    lax.fori_loop(0, NCH, chunk_body, 0)

    for k in range(NB):
        gi, di, sv, rows, msem, gsem, ssem = bufs[k]
        pltpu.make_async_copy(rows, acc_sm.at[di], ssem).wait()

    plsc.subcore_barrier()
    ob = pl.multiple_of(sid * 624, 8)

    @pl.when(sid < NS - 1)
    def _():
        pltpu.sync_copy(acc_sm.at[pl.ds(ob, 624)],
                        out_hbm.at[cid, pl.ds(ob, 624)])

    @pl.when(sid == NS - 1)
    def _():
        pltpu.sync_copy(acc_sm.at[pl.ds(ob, 640)],
                        out_hbm.at[cid, pl.ds(ob, 640)])


def _sc_scatter(y, gi, dst, s, D):
    ring_bufs = [
        pltpu.VMEM((CHUNK,), jnp.int32),
        pltpu.VMEM((CHUNK,), jnp.int32),
        pltpu.VMEM((CHUNK,), jnp.float32),
        pltpu.VMEM((CHUNK, D), jnp.float32),
    ] * 4
    sems = [pltpu.SemaphoreType.DMA] * 12
    return pl.kernel(
        functools.partial(_scatter_body, D),
        out_type=jax.ShapeDtypeStruct((NC, N_NODES, D), jnp.float32),
        mesh=_mesh,
        compiler_params=_sc_params,
        scratch_types=ring_bufs + sems + [
            pltpu.VMEM_SHARED((N_NODES, D), jnp.float32),
        ],
    )(y, gi, dst, s)


def _reduce_body(y_hbm, gi_hbm, s_hbm, out_hbm,
                 gi1, s1, rows1, gi2, s2, rows2, gi3, s3, rows3,
                 gi4, s4, rows4, gi5, s5, rows5, gi6, s6, rows6,
                 msem1, gsem1, msem2, gsem2, msem3, gsem3,
                 msem4, gsem4, msem5, gsem5, msem6, gsem6, acc8):
    cid = lax.axis_index("c")
    sid = lax.axis_index("s")
    wid = cid * NS + sid

    def za(i, _):
        for q in range(8):
            acc8[i, pl.ds(q * 16, 16)] = _Z16()
        return 0
    lax.fori_loop(0, 8, za, 0)

    ebase = wid * E_TILE
    bufs = ((gi1, s1, rows1, msem1, gsem1), (gi2, s2, rows2, msem2, gsem2),
            (gi3, s3, rows3, msem3, gsem3), (gi4, s4, rows4, msem4, gsem4),
            (gi5, s5, rows5, msem5, gsem5), (gi6, s6, rows6, msem6, gsem6))
    NB = 6
    NCH = E_TILE // CHUNK

    def meta_issue(c, buf):
        gi, sv, rows, msem, gsem = buf
        base = ebase + c * CHUNK
        pltpu.async_copy(gi_hbm.at[pl.ds(base, CHUNK)], gi, msem)
        pltpu.async_copy(s_hbm.at[pl.ds(base, CHUNK)], sv, msem)

    def gather_issue(c, buf):
        gi, sv, rows, msem, gsem = buf
        base = ebase + c * CHUNK
        pltpu.make_async_copy(gi_hbm.at[pl.ds(base, CHUNK)], gi, msem).wait()
        pltpu.make_async_copy(s_hbm.at[pl.ds(base, CHUNK)], sv, msem).wait()
        pltpu.async_copy(y_hbm.at[gi], rows, gsem)

    def consume(buf):
        gi, sv, rows, msem, gsem = buf
        pltpu.make_async_copy(y_hbm.at[gi], rows, gsem).wait()

        def rbody(j, acc):
            sj = plsc.load_gather(sv, [jnp.full((16,), j, jnp.int32)])
            return tuple(acc[q] + rows[j, pl.ds(q * 16, 16)] * sj
                         for q in range(8))
        acc = lax.fori_loop(0, CHUNK, rbody, tuple(_Z16() for _ in range(8)))
        for q in range(8):
            acc8[0, pl.ds(q * 16, 16)] = acc8[0, pl.ds(q * 16, 16)] + acc[q]

    for n in range(4):
        meta_issue(n, bufs[n])
    gather_issue(0, bufs[0])
    gather_issue(1, bufs[1])

    def chunk_body(c, _):
        n4 = c + 4
        for k in range(NB):
            @pl.when((n4 < NCH) & (n4 % NB == k))
            def _(k=k, n4=n4):
                meta_issue(n4, bufs[k])

        n2 = c + 2
        for k in range(NB):
            @pl.when((n2 < NCH) & (n2 % NB == k))
            def _(k=k, n2=n2):
                gather_issue(n2, bufs[k])

        for k in range(NB):
            @pl.when(c % NB == k)
            def _(k=k):
                consume(bufs[k])
        return 0
    lax.fori_loop(0, NCH, chunk_body, 0)

    pltpu.sync_copy(acc8, out_hbm.at[cid, sid])


def _sc_reduce(y, gi, s):
    ring = [
        pltpu.VMEM((CHUNK,), jnp.int32),
        pltpu.VMEM((CHUNK,), jnp.float32),
        pltpu.VMEM((CHUNK, D_HID), jnp.float32),
    ] * 6
    sems = [pltpu.SemaphoreType.DMA] * 12
    return pl.kernel(
        _reduce_body,
        out_type=jax.ShapeDtypeStruct((NC, NS, 8, D_HID), jnp.float32),
        mesh=_mesh,
        compiler_params=_sc_params,
        scratch_types=ring + sems + [pltpu.VMEM((8, D_HID), jnp.float32)],
    )(y, gi, s)


def _tc1_body(x_ref, w_ref, root_ref, b_ref, y_ref, h0_ref):
    xb = x_ref[...]
    for r in range(N_REL):
        y_ref[:, r, :] = jnp.dot(xb, w_ref[r], preferred_element_type=jnp.float32)
    h0_ref[...] = (jnp.dot(xb, root_ref[...], preferred_element_type=jnp.float32)
                   + b_ref[...])


def _tc_transform1(x, W, root, b, d_in, d_out):
    blk = 400
    grid = (N_NODES // blk,)
    return pl.pallas_call(
        _tc1_body,
        grid=grid,
        in_specs=[
            pl.BlockSpec((blk, d_in), lambda i: (i, 0)),
            pl.BlockSpec((N_REL, d_in, d_out), lambda i: (0, 0, 0)),
            pl.BlockSpec((d_in, d_out), lambda i: (0, 0)),
            pl.BlockSpec((1, d_out), lambda i: (0, 0)),
        ],
        out_specs=[
            pl.BlockSpec((blk, N_REL, d_out), lambda i: (i, 0, 0)),
            pl.BlockSpec((blk, d_out), lambda i: (i, 0)),
        ],
        out_shape=[
            jax.ShapeDtypeStruct((N_NODES, N_REL, d_out), jnp.float32),
            jax.ShapeDtypeStruct((N_NODES, d_out), jnp.float32),
        ],
        compiler_params=pltpu.CompilerParams(
            dimension_semantics=("parallel",)),
    )(x, W, root, b)


def _tc2_body(h0_ref, pa_ref, pb_ref, w_ref, y_ref, hs_ref):
    h = jnp.maximum(h0_ref[...] + pa_ref[...] + pb_ref[...], 0.0)
    for r in range(N_REL):
        y_ref[:, r, :] = jnp.dot(h, w_ref[r], preferred_element_type=jnp.float32)
    hs_ref[...] = jnp.sum(h, axis=0, keepdims=True)[None]


def _tc_transform2(h0, pa, pb, W, d_in, d_w):
    blk = 400
    grid = (N_NODES // blk,)
    return pl.pallas_call(
        _tc2_body,
        grid=grid,
        in_specs=[
            pl.BlockSpec((blk, d_in), lambda i: (i, 0)),
            pl.BlockSpec((blk, d_in), lambda i: (i, 0)),
            pl.BlockSpec((blk, d_in), lambda i: (i, 0)),
            pl.BlockSpec((N_REL, d_in, d_w), lambda i: (0, 0, 0)),
        ],
        out_specs=[
            pl.BlockSpec((blk, N_REL, d_w), lambda i: (i, 0, 0)),
            pl.BlockSpec((1, 1, d_in), lambda i: (i, 0, 0)),
        ],
        out_shape=[
            jax.ShapeDtypeStruct((N_NODES, N_REL, d_w), jnp.float32),
            jax.ShapeDtypeStruct((N_NODES // blk, 1, d_in), jnp.float32),
        ],
        compiler_params=pltpu.CompilerParams(
            dimension_semantics=("parallel",)),
    )(h0, pa, pb, W)


def _fin_body(hs_ref, root_ref, b_ref, ep_ref, o_ref):
    hv = jnp.sum(hs_ref[...], axis=(0, 1))[None, :]
    ep = jnp.sum(ep_ref[...], axis=(0, 1, 2))[None, :D_OUT]
    g = (jnp.dot(hv, root_ref[...], preferred_element_type=jnp.float32)
         + ep) / N_NODES + b_ref[...]
    m = jnp.max(g)
    lse = jnp.log(jnp.sum(jnp.exp(g - m))) + m
    o_ref[...] = g - lse


def _tc_finish(hsum, root, b, ep):
    nb = hsum.shape[0]
    return pl.pallas_call(
        _fin_body,
        in_specs=[
            pl.BlockSpec((nb, 1, D_HID), lambda: (0, 0, 0)),
            pl.BlockSpec((D_HID, D_OUT), lambda: (0, 0)),
            pl.BlockSpec((1, D_OUT), lambda: (0, 0)),
            pl.BlockSpec((NC, NS, 8, D_HID), lambda: (0, 0, 0, 0)),
        ],
        out_specs=pl.BlockSpec((1, D_OUT), lambda: (0, 0)),
        out_shape=jax.ShapeDtypeStruct((1, D_OUT), jnp.float32),
    )(hsum, root, b, ep)


def kernel(x, edge_index, edge_type, W1, root1, b1, W2, root2, b2):
    src = edge_index[0]
    dst = edge_index[1]
    et = edge_type

    s, gi = _sc_scales(et, dst, src)

    y1, h0 = _tc_transform1(x, W1, root1, b1.reshape(1, D_HID), D_IN, D_HID)
    p1 = _sc_scatter(y1.reshape(N_NODES * N_REL, D_HID), gi, dst, s, D_HID)

    W2p = jnp.pad(W2, ((0, 0), (0, 0), (0, D_HID - D_OUT)))
    y2, hsum = _tc_transform2(h0, p1[0], p1[1], W2p, D_HID, D_HID)
    ep = _sc_reduce(y2.reshape(N_NODES * N_REL, D_HID), gi, s)

    return _tc_finish(hsum, root2, b2.reshape(1, D_OUT), ep)


# speedup vs baseline: 47.7988x; 1.0313x over previous
import functools

import jax
import jax.numpy as jnp
from jax import lax
from jax.experimental import pallas as pl
from jax.experimental.pallas import tpu as pltpu
from jax.experimental.pallas import tpu_sc as plsc

N_NODES = 10000
N_EDGES = 320000
D_IN = 128
D_HID = 128
D_OUT = 64
N_REL = 8

NC, NS = 2, 16
NW = NC * NS
E_TILE = N_EDGES // NW
E_CNT = N_EDGES // NS
KEYS = N_NODES * N_REL
KEYS_PAD = 81920
RED = KEYS_PAD // NS
CHUNK = 80
BLK = 2000
ROWS_TILE = N_NODES // NS

_mesh = plsc.VectorSubcoreMesh(core_axis_name="c", subcore_axis_name="s")
_sc_params = pltpu.CompilerParams(needs_layout_passes=False)

_Z16 = functools.partial(jnp.zeros, (16,), jnp.float32)
_O16 = functools.partial(jnp.ones, (16,), jnp.float32)


HROWS = KEYS_PAD // 128


def _scales_body(et_hbm, dst_hbm, src_hbm, s_hbm, gi_hbm,
                 hist2, tb0, db0, sr0, tb1, db1, sr1, sb0, gb0, sb1, gb1,
                 zbuf, ridx, isem0, isem1, osem0, osem1, acc_sm):
    cid = lax.axis_index("c")
    sid = lax.axis_index("s")
    ins = ((tb0, db0, sr0, isem0, sb0, gb0, osem0),
           (tb1, db1, sr1, isem1, sb1, gb1, osem1))

    def stage_a(blk, st):
        tb, db, sr, isem = st[:4]
        base = sid * E_CNT + blk * BLK
        pltpu.async_copy(et_hbm.at[pl.ds(base, BLK)], tb, isem)
        pltpu.async_copy(dst_hbm.at[pl.ds(base, BLK)], db, isem)

    def wait_a(blk, st):
        tb, db, sr, isem = st[:4]
        base = sid * E_CNT + blk * BLK
        pltpu.make_async_copy(et_hbm.at[pl.ds(base, BLK)], tb, isem).wait()
        pltpu.make_async_copy(dst_hbm.at[pl.ds(base, BLK)], db, isem).wait()

    stage_a(0, ins[0])
    stage_a(1, ins[1])

    def zh(i, _):
        for q in range(8):
            hist2[i, pl.ds(q * 16, 16)] = _Z16()
        return 0
    lax.fori_loop(0, HROWS, zh, 0)
    def zz(i, _):
        for q in range(8):
            zbuf[i, pl.ds(q * 16, 16)] = _Z16()
        return 0
    lax.fori_loop(0, HROWS // NS, zz, 0)
    pltpu.sync_copy(zbuf, acc_sm.at[pl.ds(sid * (HROWS // NS), HROWS // NS)])
    plsc.subcore_barrier()

    NA = E_CNT // BLK
    for blk in range(NA):
        st = ins[blk % 2]
        wait_a(blk, st)

        def cbody(i, _):
            t = st[0][pl.ds(i * 16, 16)]
            d = st[1][pl.ds(i * 16, 16)]
            key = d * N_REL + t
            plsc.addupdate_scatter(
                hist2, [key >> 7, key & 127], _O16())
            return 0
        lax.fori_loop(0, BLK // 16, cbody, 0)
        if blk + 2 < NA:
            stage_a(blk + 2, st)

    for k in range(HROWS // 128):
        for j in range(8):
            ridx[pl.ds(j * 16, 16)] = (
                lax.iota(jnp.int32, 16) + (k * 128 + j * 16))
        pltpu.sync_copy(hist2.at[pl.ds(k * 128, 128)],
                        acc_sm.at[ridx], add=True)
    plsc.subcore_barrier()

    ebase = (cid * NS + sid) * E_TILE

    def stage_c(blk, st):
        tb, db, sr, isem = st[:4]
        base = ebase + blk * BLK
        pltpu.async_copy(et_hbm.at[pl.ds(base, BLK)], tb, isem)
        pltpu.async_copy(dst_hbm.at[pl.ds(base, BLK)], db, isem)
        pltpu.async_copy(src_hbm.at[pl.ds(base, BLK)], sr, isem)

    def wait_c(blk, st):
        tb, db, sr, isem = st[:4]
        base = ebase + blk * BLK
        pltpu.make_async_copy(et_hbm.at[pl.ds(base, BLK)], tb, isem).wait()
        pltpu.make_async_copy(dst_hbm.at[pl.ds(base, BLK)], db, isem).wait()
        pltpu.make_async_copy(src_hbm.at[pl.ds(base, BLK)], sr, isem).wait()

    def wait_out(blk, st):
        sb, gb, osem = st[4:]
        base = ebase + blk * BLK
        pltpu.make_async_copy(sb, s_hbm.at[pl.ds(base, BLK)], osem).wait()
        pltpu.make_async_copy(gb, gi_hbm.at[pl.ds(base, BLK)], osem).wait()

    pltpu.sync_copy(acc_sm, hist2)
    NCB = E_TILE // BLK
    stage_c(0, ins[0])
    stage_c(1, ins[1])
    for blk in range(NCB):
        st = ins[blk % 2]
        tb, db, sr, isem, sb, gb, osem = st
        wait_c(blk, st)
        if blk >= 2:
            wait_out(blk - 2, st)

        @plsc.parallel_loop(0, BLK // 16, 1, unroll=4)
        def _(i):
            t = tb[pl.ds(i * 16, 16)]
            d = db[pl.ds(i * 16, 16)]
            s_ = sr[pl.ds(i * 16, 16)]
            key = d * N_REL + t
            c = plsc.load_gather(hist2, [key >> 7, key & 127])
            sb[pl.ds(i * 16, 16)] = 1.0 / jnp.maximum(c, 1.0)
            gb[pl.ds(i * 16, 16)] = s_ * N_REL + t
        base = ebase + blk * BLK
        pltpu.async_copy(sb, s_hbm.at[pl.ds(base, BLK)], osem)
        pltpu.async_copy(gb, gi_hbm.at[pl.ds(base, BLK)], osem)
        if blk + 2 < NCB:
            stage_c(blk + 2, st)
    wait_out(NCB - 2, ins[(NCB - 2) % 2])
    wait_out(NCB - 1, ins[(NCB - 1) % 2])


def _sc_scales(et, dst, src):
    return pl.kernel(
        _scales_body,
        out_type=[jax.ShapeDtypeStruct((N_EDGES,), jnp.float32),
                  jax.ShapeDtypeStruct((N_EDGES,), jnp.int32)],
        mesh=_mesh,
        compiler_params=_sc_params,
        scratch_types=[
            pltpu.VMEM((HROWS, 128), jnp.float32),
            pltpu.VMEM((BLK,), jnp.int32),
            pltpu.VMEM((BLK,), jnp.int32),
            pltpu.VMEM((BLK,), jnp.int32),
            pltpu.VMEM((BLK,), jnp.int32),
            pltpu.VMEM((BLK,), jnp.int32),
            pltpu.VMEM((BLK,), jnp.int32),
            pltpu.VMEM((BLK,), jnp.float32),
            pltpu.VMEM((BLK,), jnp.int32),
            pltpu.VMEM((BLK,), jnp.float32),
            pltpu.VMEM((BLK,), jnp.int32),
            pltpu.VMEM((HROWS // NS, 128), jnp.float32),
            pltpu.VMEM((128,), jnp.int32),
            pltpu.SemaphoreType.DMA,
            pltpu.SemaphoreType.DMA,
            pltpu.SemaphoreType.DMA,
            pltpu.SemaphoreType.DMA,
            pltpu.VMEM_SHARED((HROWS, 128), jnp.float32),
        ],
    )(et, dst, src)


def _scatter_body(D, y_hbm, gi_hbm, di_hbm, s_hbm, out_hbm,
                  gi1, di1, s1, rows1, gi2, di2, s2, rows2,
                  gi3, di3, s3, rows3, gi4, di4, s4, rows4,
                  msem1, gsem1, ssem1, msem2, gsem2, ssem2,
                  msem3, gsem3, ssem3, msem4, gsem4, ssem4, acc_sm):
    cid = lax.axis_index("c")
    sid = lax.axis_index("s")
    wid = cid * NS + sid
    nq = D // 16

    def zb(i, _):
        for q in range(nq):
            rows1[i, pl.ds(q * 16, 16)] = _Z16()
        return 0
    lax.fori_loop(0, CHUNK, zb, 0)
    rbase = sid * ROWS_TILE
    for q in range(ROWS_TILE // CHUNK):
        pltpu.async_copy(rows1, acc_sm.at[pl.ds(rbase + q * CHUNK, CHUNK)],
                         msem1)
    rem = ROWS_TILE % CHUNK
    if rem:
        pltpu.async_copy(rows1.at[pl.ds(0, rem)],
                         acc_sm.at[pl.ds(rbase + ROWS_TILE - rem, rem)],
                         msem1)
    for q in range(ROWS_TILE // CHUNK):
        pltpu.make_async_copy(
            rows1, acc_sm.at[pl.ds(rbase + q * CHUNK, CHUNK)], msem1).wait()
    if rem:
        pltpu.make_async_copy(
            rows1.at[pl.ds(0, rem)],
            acc_sm.at[pl.ds(rbase + ROWS_TILE - rem, rem)], msem1).wait()
    plsc.subcore_barrier()

    ebase = wid * E_TILE
    bufs = ((gi1, di1, s1, rows1, msem1, gsem1, ssem1),
            (gi2, di2, s2, rows2, msem2, gsem2, ssem2),
            (gi3, di3, s3, rows3, msem3, gsem3, ssem3),
            (gi4, di4, s4, rows4, msem4, gsem4, ssem4))
    NB = 4
    NCH = E_TILE // CHUNK

    def meta_issue(c, buf, drain):
        gi, di, sv, rows, msem, gsem, ssem = buf
        if drain:
            pltpu.make_async_copy(rows, acc_sm.at[di], ssem).wait()
        base = ebase + c * CHUNK
        pltpu.async_copy(gi_hbm.at[pl.ds(base, CHUNK)], gi, msem)
        pltpu.async_copy(di_hbm.at[pl.ds(base, CHUNK)], di, msem)
        pltpu.async_copy(s_hbm.at[pl.ds(base, CHUNK)], sv, msem)

    def gather_issue(c, buf):
        gi, di, sv, rows, msem, gsem, ssem = buf
        base = ebase + c * CHUNK
        pltpu.make_async_copy(gi_hbm.at[pl.ds(base, CHUNK)], gi, msem).wait()
        pltpu.make_async_copy(di_hbm.at[pl.ds(base, CHUNK)], di, msem).wait()
        pltpu.make_async_copy(s_hbm.at[pl.ds(base, CHUNK)], sv, msem).wait()
        pltpu.async_copy(y_hbm.at[gi], rows, gsem)

    def consume(buf):
        gi, di, sv, rows, msem, gsem, ssem = buf
        pltpu.make_async_copy(y_hbm.at[gi], rows, gsem).wait()

        @plsc.parallel_loop(0, CHUNK, 1, unroll=4)
        def _(j):
            sj = plsc.load_gather(sv, [jnp.full((16,), j, jnp.int32)])
            for q in range(nq):
                rows[j, pl.ds(q * 16, 16)] = rows[j, pl.ds(q * 16, 16)] * sj
        pltpu.async_copy(rows, acc_sm.at[di], ssem, add=True)

    meta_issue(0, bufs[0], drain=False)
    meta_issue(1, bufs[1], drain=False)
    gather_issue(0, bufs[0])

    def chunk_body(c, _):
        n2 = c + 2
        for k in range(NB):
            @pl.when((n2 < NCH) & (n2 % NB == k) & (c >= 2))
            def _(k=k, n2=n2):
                meta_issue(n2, bufs[k], drain=True)

            @pl.when((n2 < NCH) & (n2 % NB == k) & (c < 2))
            def _(k=k, n2=n2):
                meta_issue(n2, bufs[k], drain=False)

        n1 = c + 1
        for k in range(NB):
            @pl.when((n1 < NCH) & (n1 % NB == k))
            def _(k=k, n1=n1):
                gather_issue(n1, bufs[k])

        for k in range(NB):
            @pl.when(c % NB == k)
            def _(k=k):
                consume(bufs[k])
        return 0
    lax.fori_loop(0, NCH, chunk_body, 0)

    for k in range(NB):
        gi, di, sv, rows, msem, gsem, ssem = bufs[k]
        pltpu.make_async_copy(rows, acc_sm.at[di], ssem).wait()

    plsc.subcore_barrier()
    ob = pl.multiple_of(sid * 624, 8)

    @pl.when(sid < NS - 1)
    def _():
        pltpu.sync_copy(acc_sm.at[pl.ds(ob, 624)],
                        out_hbm.at[cid, pl.ds(ob, 624)])

    @pl.when(sid == NS - 1)
    def _():
        pltpu.sync_copy(acc_sm.at[pl.ds(ob, 640)],
                        out_hbm.at[cid, pl.ds(ob, 640)])


def _sc_scatter(y, gi, dst, s, D):
    ring_bufs = [
        pltpu.VMEM((CHUNK,), jnp.int32),
        pltpu.VMEM((CHUNK,), jnp.int32),
        pltpu.VMEM((CHUNK,), jnp.float32),
        pltpu.VMEM((CHUNK, D), jnp.float32),
    ] * 4
    sems = [pltpu.SemaphoreType.DMA] * 12
    return pl.kernel(
        functools.partial(_scatter_body, D),
        out_type=jax.ShapeDtypeStruct((NC, N_NODES, D), jnp.float32),
        mesh=_mesh,
        compiler_params=_sc_params,
        scratch_types=ring_bufs + sems + [
            pltpu.VMEM_SHARED((N_NODES, D), jnp.float32),
        ],
    )(y, gi, dst, s)


def _reduce_body(y_hbm, gi_hbm, s_hbm, out_hbm,
                 gi1, s1, rows1, gi2, s2, rows2, gi3, s3, rows3,
                 gi4, s4, rows4, gi5, s5, rows5, gi6, s6, rows6,
                 msem1, gsem1, msem2, gsem2, msem3, gsem3,
                 msem4, gsem4, msem5, gsem5, msem6, gsem6, acc8):
    cid = lax.axis_index("c")
    sid = lax.axis_index("s")
    wid = cid * NS + sid

    def za(i, _):
        for q in range(8):
            acc8[i, pl.ds(q * 16, 16)] = _Z16()
        return 0
    lax.fori_loop(0, 8, za, 0)

    ebase = wid * E_TILE
    bufs = ((gi1, s1, rows1, msem1, gsem1), (gi2, s2, rows2, msem2, gsem2),
            (gi3, s3, rows3, msem3, gsem3), (gi4, s4, rows4, msem4, gsem4),
            (gi5, s5, rows5, msem5, gsem5), (gi6, s6, rows6, msem6, gsem6))
    NB = 6
    NCH = E_TILE // CHUNK

    def meta_issue(c, buf):
        gi, sv, rows, msem, gsem = buf
        base = ebase + c * CHUNK
        pltpu.async_copy(gi_hbm.at[pl.ds(base, CHUNK)], gi, msem)
        pltpu.async_copy(s_hbm.at[pl.ds(base, CHUNK)], sv, msem)

    def gather_issue(c, buf):
        gi, sv, rows, msem, gsem = buf
        base = ebase + c * CHUNK
        pltpu.make_async_copy(gi_hbm.at[pl.ds(base, CHUNK)], gi, msem).wait()
        pltpu.make_async_copy(s_hbm.at[pl.ds(base, CHUNK)], sv, msem).wait()
        pltpu.async_copy(y_hbm.at[gi], rows, gsem)

    def consume(buf):
        gi, sv, rows, msem, gsem = buf
        pltpu.make_async_copy(y_hbm.at[gi], rows, gsem).wait()

        def rbody(j, acc):
            sj = plsc.load_gather(sv, [jnp.full((16,), j, jnp.int32)])
            return tuple(acc[q] + rows[j, pl.ds(q * 16, 16)] * sj
                         for q in range(8))
        acc = lax.fori_loop(0, CHUNK, rbody, tuple(_Z16() for _ in range(8)))
        for q in range(8):
            acc8[0, pl.ds(q * 16, 16)] = acc8[0, pl.ds(q * 16, 16)] + acc[q]

    for n in range(5):
        meta_issue(n, bufs[n])
    gather_issue(0, bufs[0])
    gather_issue(1, bufs[1])
    gather_issue(2, bufs[2])

    def chunk_body(c, _):
        n4 = c + 5
        for k in range(NB):
            @pl.when((n4 < NCH) & (n4 % NB == k))
            def _(k=k, n4=n4):
                meta_issue(n4, bufs[k])

        n2 = c + 3
        for k in range(NB):
            @pl.when((n2 < NCH) & (n2 % NB == k))
            def _(k=k, n2=n2):
                gather_issue(n2, bufs[k])

        for k in range(NB):
            @pl.when(c % NB == k)
            def _(k=k):
                consume(bufs[k])
        return 0
    lax.fori_loop(0, NCH, chunk_body, 0)

    pltpu.sync_copy(acc8, out_hbm.at[cid, sid])


def _sc_reduce(y, gi, s):
    ring = [
        pltpu.VMEM((CHUNK,), jnp.int32),
        pltpu.VMEM((CHUNK,), jnp.float32),
        pltpu.VMEM((CHUNK, D_HID), jnp.float32),
    ] * 6
    sems = [pltpu.SemaphoreType.DMA] * 12
    return pl.kernel(
        _reduce_body,
        out_type=jax.ShapeDtypeStruct((NC, NS, 8, D_HID), jnp.float32),
        mesh=_mesh,
        compiler_params=_sc_params,
        scratch_types=ring + sems + [pltpu.VMEM((8, D_HID), jnp.float32)],
    )(y, gi, s)


def _tc1_body(x_ref, w_ref, root_ref, b_ref, y_ref, h0_ref):
    xb = x_ref[...]
    for r in range(N_REL):
        y_ref[:, r, :] = jnp.dot(xb, w_ref[r], preferred_element_type=jnp.float32)
    h0_ref[...] = (jnp.dot(xb, root_ref[...], preferred_element_type=jnp.float32)
                   + b_ref[...])


def _tc_transform1(x, W, root, b, d_in, d_out):
    blk = 400
    grid = (N_NODES // blk,)
    return pl.pallas_call(
        _tc1_body,
        grid=grid,
        in_specs=[
            pl.BlockSpec((blk, d_in), lambda i: (i, 0)),
            pl.BlockSpec((N_REL, d_in, d_out), lambda i: (0, 0, 0)),
            pl.BlockSpec((d_in, d_out), lambda i: (0, 0)),
            pl.BlockSpec((1, d_out), lambda i: (0, 0)),
        ],
        out_specs=[
            pl.BlockSpec((blk, N_REL, d_out), lambda i: (i, 0, 0)),
            pl.BlockSpec((blk, d_out), lambda i: (i, 0)),
        ],
        out_shape=[
            jax.ShapeDtypeStruct((N_NODES, N_REL, d_out), jnp.float32),
            jax.ShapeDtypeStruct((N_NODES, d_out), jnp.float32),
        ],
        compiler_params=pltpu.CompilerParams(
            dimension_semantics=("parallel",)),
    )(x, W, root, b)


def _tc2_body(h0_ref, pa_ref, pb_ref, w_ref, y_ref, hs_ref):
    h = jnp.maximum(h0_ref[...] + pa_ref[...] + pb_ref[...], 0.0)
    for r in range(N_REL):
        y_ref[:, r, :] = jnp.dot(h, w_ref[r], preferred_element_type=jnp.float32)
    hs_ref[...] = jnp.sum(h, axis=0, keepdims=True)[None]


def _tc_transform2(h0, pa, pb, W, d_in, d_w):
    blk = 400
    grid = (N_NODES // blk,)
    return pl.pallas_call(
        _tc2_body,
        grid=grid,
        in_specs=[
            pl.BlockSpec((blk, d_in), lambda i: (i, 0)),
            pl.BlockSpec((blk, d_in), lambda i: (i, 0)),
            pl.BlockSpec((blk, d_in), lambda i: (i, 0)),
            pl.BlockSpec((N_REL, d_in, d_w), lambda i: (0, 0, 0)),
        ],
        out_specs=[
            pl.BlockSpec((blk, N_REL, d_w), lambda i: (i, 0, 0)),
            pl.BlockSpec((1, 1, d_in), lambda i: (i, 0, 0)),
        ],
        out_shape=[
            jax.ShapeDtypeStruct((N_NODES, N_REL, d_w), jnp.float32),
            jax.ShapeDtypeStruct((N_NODES // blk, 1, d_in), jnp.float32),
        ],
        compiler_params=pltpu.CompilerParams(
            dimension_semantics=("parallel",)),
    )(h0, pa, pb, W)


def _fin_body(hs_ref, root_ref, b_ref, ep_ref, o_ref):
    hv = jnp.sum(hs_ref[...], axis=(0, 1))[None, :]
    ep = jnp.sum(ep_ref[...], axis=(0, 1, 2))[None, :D_OUT]
    g = (jnp.dot(hv, root_ref[...], preferred_element_type=jnp.float32)
         + ep) / N_NODES + b_ref[...]
    m = jnp.max(g)
    lse = jnp.log(jnp.sum(jnp.exp(g - m))) + m
    o_ref[...] = g - lse


def _tc_finish(hsum, root, b, ep):
    nb = hsum.shape[0]
    return pl.pallas_call(
        _fin_body,
        in_specs=[
            pl.BlockSpec((nb, 1, D_HID), lambda: (0, 0, 0)),
            pl.BlockSpec((D_HID, D_OUT), lambda: (0, 0)),
            pl.BlockSpec((1, D_OUT), lambda: (0, 0)),
            pl.BlockSpec((NC, NS, 8, D_HID), lambda: (0, 0, 0, 0)),
        ],
        out_specs=pl.BlockSpec((1, D_OUT), lambda: (0, 0)),
        out_shape=jax.ShapeDtypeStruct((1, D_OUT), jnp.float32),
    )(hsum, root, b, ep)


def kernel(x, edge_index, edge_type, W1, root1, b1, W2, root2, b2):
    src = edge_index[0]
    dst = edge_index[1]
    et = edge_type

    s, gi = _sc_scales(et, dst, src)

    y1, h0 = _tc_transform1(x, W1, root1, b1.reshape(1, D_HID), D_IN, D_HID)
    p1 = _sc_scatter(y1.reshape(N_NODES * N_REL, D_HID), gi, dst, s, D_HID)

    W2p = jnp.pad(W2, ((0, 0), (0, 0), (0, D_HID - D_OUT)))
    y2, hsum = _tc_transform2(h0, p1[0], p1[1], W2p, D_HID, D_HID)
    ep = _sc_reduce(y2.reshape(N_NODES * N_REL, D_HID), gi, s)

    return _tc_finish(hsum, root2, b2.reshape(1, D_OUT), ep)


# speedup vs baseline: 48.2038x; 1.0085x over previous
import functools

import jax
import jax.numpy as jnp
from jax import lax
from jax.experimental import pallas as pl
from jax.experimental.pallas import tpu as pltpu
from jax.experimental.pallas import tpu_sc as plsc

N_NODES = 10000
N_EDGES = 320000
D_IN = 128
D_HID = 128
D_OUT = 64
N_REL = 8

NC, NS = 2, 16
NW = NC * NS
E_TILE = N_EDGES // NW
E_CNT = N_EDGES // NS
KEYS = N_NODES * N_REL
KEYS_PAD = 81920
RED = KEYS_PAD // NS
CHUNK = 80
BLK = 2000
ROWS_TILE = N_NODES // NS

_mesh = plsc.VectorSubcoreMesh(core_axis_name="c", subcore_axis_name="s")
_sc_params = pltpu.CompilerParams(needs_layout_passes=False)

_Z16 = functools.partial(jnp.zeros, (16,), jnp.float32)
_O16 = functools.partial(jnp.ones, (16,), jnp.float32)


HROWS = KEYS_PAD // 128


def _scales_body(et_hbm, dst_hbm, src_hbm, s_hbm, gi_hbm,
                 hist2, tb0, db0, sr0, tb1, db1, sr1, sb0, gb0, sb1, gb1,
                 zbuf, ridx, isem0, isem1, osem0, osem1, acc_sm):
    cid = lax.axis_index("c")
    sid = lax.axis_index("s")
    ins = ((tb0, db0, sr0, isem0, sb0, gb0, osem0),
           (tb1, db1, sr1, isem1, sb1, gb1, osem1))

    def stage_a(blk, st):
        tb, db, sr, isem = st[:4]
        base = sid * E_CNT + blk * BLK
        pltpu.async_copy(et_hbm.at[pl.ds(base, BLK)], tb, isem)
        pltpu.async_copy(dst_hbm.at[pl.ds(base, BLK)], db, isem)

    def wait_a(blk, st):
        tb, db, sr, isem = st[:4]
        base = sid * E_CNT + blk * BLK
        pltpu.make_async_copy(et_hbm.at[pl.ds(base, BLK)], tb, isem).wait()
        pltpu.make_async_copy(dst_hbm.at[pl.ds(base, BLK)], db, isem).wait()

    stage_a(0, ins[0])
    stage_a(1, ins[1])

    def zh(i, _):
        for q in range(8):
            hist2[i, pl.ds(q * 16, 16)] = _Z16()
        return 0
    lax.fori_loop(0, HROWS, zh, 0)
    def zz(i, _):
        for q in range(8):
            zbuf[i, pl.ds(q * 16, 16)] = _Z16()
        return 0
    lax.fori_loop(0, HROWS // NS, zz, 0)
    pltpu.sync_copy(zbuf, acc_sm.at[pl.ds(sid * (HROWS // NS), HROWS // NS)])
    plsc.subcore_barrier()

    NA = E_CNT // BLK
    for blk in range(NA):
        st = ins[blk % 2]
        wait_a(blk, st)

        def cbody(i, _):
            t = st[0][pl.ds(i * 16, 16)]
            d = st[1][pl.ds(i * 16, 16)]
            key = d * N_REL + t
            plsc.addupdate_scatter(
                hist2, [key >> 7, key & 127], _O16())
            return 0
        lax.fori_loop(0, BLK // 16, cbody, 0)
        if blk + 2 < NA:
            stage_a(blk + 2, st)

    for k in range(HROWS // 128):
        for j in range(8):
            ridx[pl.ds(j * 16, 16)] = (
                lax.iota(jnp.int32, 16) + (k * 128 + j * 16))
        pltpu.sync_copy(hist2.at[pl.ds(k * 128, 128)],
                        acc_sm.at[ridx], add=True)
    plsc.subcore_barrier()

    ebase = (cid * NS + sid) * E_TILE

    def stage_c(blk, st):
        tb, db, sr, isem = st[:4]
        base = ebase + blk * BLK
        pltpu.async_copy(et_hbm.at[pl.ds(base, BLK)], tb, isem)
        pltpu.async_copy(dst_hbm.at[pl.ds(base, BLK)], db, isem)
        pltpu.async_copy(src_hbm.at[pl.ds(base, BLK)], sr, isem)

    def wait_c(blk, st):
        tb, db, sr, isem = st[:4]
        base = ebase + blk * BLK
        pltpu.make_async_copy(et_hbm.at[pl.ds(base, BLK)], tb, isem).wait()
        pltpu.make_async_copy(dst_hbm.at[pl.ds(base, BLK)], db, isem).wait()
        pltpu.make_async_copy(src_hbm.at[pl.ds(base, BLK)], sr, isem).wait()

    def wait_out(blk, st):
        sb, gb, osem = st[4:]
        base = ebase + blk * BLK
        pltpu.make_async_copy(sb, s_hbm.at[pl.ds(base, BLK)], osem).wait()
        pltpu.make_async_copy(gb, gi_hbm.at[pl.ds(base, BLK)], osem).wait()

    pltpu.sync_copy(acc_sm, hist2)
    NCB = E_TILE // BLK
    stage_c(0, ins[0])
    stage_c(1, ins[1])
    for blk in range(NCB):
        st = ins[blk % 2]
        tb, db, sr, isem, sb, gb, osem = st
        wait_c(blk, st)
        if blk >= 2:
            wait_out(blk - 2, st)

        @plsc.parallel_loop(0, BLK // 16, 1, unroll=4)
        def _(i):
            t = tb[pl.ds(i * 16, 16)]
            d = db[pl.ds(i * 16, 16)]
            s_ = sr[pl.ds(i * 16, 16)]
            key = d * N_REL + t
            c = plsc.load_gather(hist2, [key >> 7, key & 127])
            sb[pl.ds(i * 16, 16)] = 1.0 / jnp.maximum(c, 1.0)
            gb[pl.ds(i * 16, 16)] = s_ * N_REL + t
        base = ebase + blk * BLK
        pltpu.async_copy(sb, s_hbm.at[pl.ds(base, BLK)], osem)
        pltpu.async_copy(gb, gi_hbm.at[pl.ds(base, BLK)], osem)
        if blk + 2 < NCB:
            stage_c(blk + 2, st)
    wait_out(NCB - 2, ins[(NCB - 2) % 2])
    wait_out(NCB - 1, ins[(NCB - 1) % 2])


def _sc_scales(et, dst, src):
    return pl.kernel(
        _scales_body,
        out_type=[jax.ShapeDtypeStruct((N_EDGES,), jnp.float32),
                  jax.ShapeDtypeStruct((N_EDGES,), jnp.int32)],
        mesh=_mesh,
        compiler_params=_sc_params,
        scratch_types=[
            pltpu.VMEM((HROWS, 128), jnp.float32),
            pltpu.VMEM((BLK,), jnp.int32),
            pltpu.VMEM((BLK,), jnp.int32),
            pltpu.VMEM((BLK,), jnp.int32),
            pltpu.VMEM((BLK,), jnp.int32),
            pltpu.VMEM((BLK,), jnp.int32),
            pltpu.VMEM((BLK,), jnp.int32),
            pltpu.VMEM((BLK,), jnp.float32),
            pltpu.VMEM((BLK,), jnp.int32),
            pltpu.VMEM((BLK,), jnp.float32),
            pltpu.VMEM((BLK,), jnp.int32),
            pltpu.VMEM((HROWS // NS, 128), jnp.float32),
            pltpu.VMEM((128,), jnp.int32),
            pltpu.SemaphoreType.DMA,
            pltpu.SemaphoreType.DMA,
            pltpu.SemaphoreType.DMA,
            pltpu.SemaphoreType.DMA,
            pltpu.VMEM_SHARED((HROWS, 128), jnp.float32),
        ],
    )(et, dst, src)


def _scatter_body(D, y_hbm, gi_hbm, di_hbm, s_hbm, out_hbm,
                  gi1, di1, s1, rows1, gi2, di2, s2, rows2,
                  gi3, di3, s3, rows3, gi4, di4, s4, rows4,
                  msem1, gsem1, ssem1, msem2, gsem2, ssem2,
                  msem3, gsem3, ssem3, msem4, gsem4, ssem4, acc_sm):
    cid = lax.axis_index("c")
    sid = lax.axis_index("s")
    wid = cid * NS + sid
    nq = D // 16

    def zb(i, _):
        for q in range(nq):
            rows1[i, pl.ds(q * 16, 16)] = _Z16()
        return 0
    lax.fori_loop(0, CHUNK, zb, 0)
    rbase = sid * ROWS_TILE
    for q in range(ROWS_TILE // CHUNK):
        pltpu.async_copy(rows1, acc_sm.at[pl.ds(rbase + q * CHUNK, CHUNK)],
                         msem1)
    rem = ROWS_TILE % CHUNK
    if rem:
        pltpu.async_copy(rows1.at[pl.ds(0, rem)],
                         acc_sm.at[pl.ds(rbase + ROWS_TILE - rem, rem)],
                         msem1)
    for q in range(ROWS_TILE // CHUNK):
        pltpu.make_async_copy(
            rows1, acc_sm.at[pl.ds(rbase + q * CHUNK, CHUNK)], msem1).wait()
    if rem:
        pltpu.make_async_copy(
            rows1.at[pl.ds(0, rem)],
            acc_sm.at[pl.ds(rbase + ROWS_TILE - rem, rem)], msem1).wait()
    plsc.subcore_barrier()

    ebase = wid * E_TILE
    bufs = ((gi1, di1, s1, rows1, msem1, gsem1, ssem1),
            (gi2, di2, s2, rows2, msem2, gsem2, ssem2),
            (gi3, di3, s3, rows3, msem3, gsem3, ssem3),
            (gi4, di4, s4, rows4, msem4, gsem4, ssem4))
    NB = 4
    NCH = E_TILE // CHUNK

    def meta_issue(c, buf, drain):
        gi, di, sv, rows, msem, gsem, ssem = buf
        if drain:
            pltpu.make_async_copy(rows, acc_sm.at[di], ssem).wait()
        base = ebase + c * CHUNK
        pltpu.async_copy(gi_hbm.at[pl.ds(base, CHUNK)], gi, msem)
        pltpu.async_copy(di_hbm.at[pl.ds(base, CHUNK)], di, msem)
        pltpu.async_copy(s_hbm.at[pl.ds(base, CHUNK)], sv, msem)

    def gather_issue(c, buf):
        gi, di, sv, rows, msem, gsem, ssem = buf
        base = ebase + c * CHUNK
        pltpu.make_async_copy(gi_hbm.at[pl.ds(base, CHUNK)], gi, msem).wait()
        pltpu.make_async_copy(di_hbm.at[pl.ds(base, CHUNK)], di, msem).wait()
        pltpu.make_async_copy(s_hbm.at[pl.ds(base, CHUNK)], sv, msem).wait()
        pltpu.async_copy(y_hbm.at[gi], rows, gsem)

    def consume(buf):
        gi, di, sv, rows, msem, gsem, ssem = buf
        pltpu.make_async_copy(y_hbm.at[gi], rows, gsem).wait()

        @plsc.parallel_loop(0, CHUNK, 1, unroll=4)
        def _(j):
            sj = plsc.load_gather(sv, [jnp.full((16,), j, jnp.int32)])
            for q in range(nq):
                rows[j, pl.ds(q * 16, 16)] = rows[j, pl.ds(q * 16, 16)] * sj
        pltpu.async_copy(rows, acc_sm.at[di], ssem, add=True)

    meta_issue(0, bufs[0], drain=False)
    meta_issue(1, bufs[1], drain=False)
    gather_issue(0, bufs[0])

    def chunk_body(c, _):
        n2 = c + 2
        for k in range(NB):
            @pl.when((n2 < NCH) & (n2 % NB == k) & (c >= 2))
            def _(k=k, n2=n2):
                meta_issue(n2, bufs[k], drain=True)

            @pl.when((n2 < NCH) & (n2 % NB == k) & (c < 2))
            def _(k=k, n2=n2):
                meta_issue(n2, bufs[k], drain=False)

        n1 = c + 1
        for k in range(NB):
            @pl.when((n1 < NCH) & (n1 % NB == k))
            def _(k=k, n1=n1):
                gather_issue(n1, bufs[k])

        for k in range(NB):
            @pl.when(c % NB == k)
            def _(k=k):
                consume(bufs[k])
        return 0
    lax.fori_loop(0, NCH, chunk_body, 0)

    for k in range(NB):
        gi, di, sv, rows, msem, gsem, ssem = bufs[k]
        pltpu.make_async_copy(rows, acc_sm.at[di], ssem).wait()

    plsc.subcore_barrier()
    ob = pl.multiple_of(sid * 624, 8)

    @pl.when(sid < NS - 1)
    def _():
        pltpu.sync_copy(acc_sm.at[pl.ds(ob, 624)],
                        out_hbm.at[cid, pl.ds(ob, 624)])

    @pl.when(sid == NS - 1)
    def _():
        pltpu.sync_copy(acc_sm.at[pl.ds(ob, 640)],
                        out_hbm.at[cid, pl.ds(ob, 640)])


def _sc_scatter(y, gi, dst, s, D):
    ring_bufs = [
        pltpu.VMEM((CHUNK,), jnp.int32),
        pltpu.VMEM((CHUNK,), jnp.int32),
        pltpu.VMEM((CHUNK,), jnp.float32),
        pltpu.VMEM((CHUNK, D), jnp.float32),
    ] * 4
    sems = [pltpu.SemaphoreType.DMA] * 12
    return pl.kernel(
        functools.partial(_scatter_body, D),
        out_type=jax.ShapeDtypeStruct((NC, N_NODES, D), jnp.float32),
        mesh=_mesh,
        compiler_params=_sc_params,
        scratch_types=ring_bufs + sems + [
            pltpu.VMEM_SHARED((N_NODES, D), jnp.float32),
        ],
    )(y, gi, dst, s)


def _reduce_body(y_hbm, gi_hbm, s_hbm, out_hbm,
                 gi1, gh1, s1, rows1, gi2, gh2, s2, rows2, gi3, gh3, s3, rows3,
                 gi4, gh4, s4, rows4, gi5, gh5, s5, rows5, gi6, gh6, s6, rows6,
                 msem1, gsem1, msem2, gsem2, msem3, gsem3,
                 msem4, gsem4, msem5, gsem5, msem6, gsem6, acc8):
    cid = lax.axis_index("c")
    sid = lax.axis_index("s")
    wid = cid * NS + sid

    def za(i, _):
        for q in range(8):
            acc8[i, pl.ds(q * 16, 16)] = _Z16()
        return 0
    lax.fori_loop(0, 8, za, 0)

    ebase = wid * E_TILE
    bufs = ((gi1, gh1, s1, rows1, msem1, gsem1),
            (gi2, gh2, s2, rows2, msem2, gsem2),
            (gi3, gh3, s3, rows3, msem3, gsem3),
            (gi4, gh4, s4, rows4, msem4, gsem4),
            (gi5, gh5, s5, rows5, msem5, gsem5),
            (gi6, gh6, s6, rows6, msem6, gsem6))
    NB = 6
    NCH = E_TILE // CHUNK

    def meta_issue(c, buf):
        gi, gh, sv, rows, msem, gsem = buf
        base = ebase + c * CHUNK
        pltpu.async_copy(gi_hbm.at[pl.ds(base, CHUNK)], gi, msem)
        pltpu.async_copy(s_hbm.at[pl.ds(base, CHUNK)], sv, msem)

    def gather_issue(c, buf):
        gi, gh, sv, rows, msem, gsem = buf
        base = ebase + c * CHUNK
        pltpu.make_async_copy(gi_hbm.at[pl.ds(base, CHUNK)], gi, msem).wait()
        pltpu.make_async_copy(s_hbm.at[pl.ds(base, CHUNK)], sv, msem).wait()
        for j in range(CHUNK // 16):
            v = gi[pl.ds(j * 16, 16)]
            gh[pl.ds(j * 16, 16)] = v & 1
            gi[pl.ds(j * 16, 16)] = v >> 1
        pltpu.async_copy(y_hbm.at[gi], rows, gsem)

    def consume(buf):
        gi, gh, sv, rows, msem, gsem = buf
        pltpu.make_async_copy(y_hbm.at[gi], rows, gsem).wait()

        def rbody(j, acc):
            idx = jnp.full((16,), j, jnp.int32)
            sj = plsc.load_gather(sv, [idx])
            hj = plsc.load_gather(gh, [idx])
            hm = hj == 1
            return tuple(
                acc[q] + jnp.where(hm,
                                   rows[j, pl.ds(64 + q * 16, 16)],
                                   rows[j, pl.ds(q * 16, 16)]) * sj
                for q in range(4))
        acc = lax.fori_loop(0, CHUNK, rbody, tuple(_Z16() for _ in range(4)))
        for q in range(4):
            acc8[0, pl.ds(q * 16, 16)] = acc8[0, pl.ds(q * 16, 16)] + acc[q]

    for n in range(5):
        meta_issue(n, bufs[n])
    gather_issue(0, bufs[0])
    gather_issue(1, bufs[1])
    gather_issue(2, bufs[2])

    def chunk_body(c, _):
        n4 = c + 5
        for k in range(NB):
            @pl.when((n4 < NCH) & (n4 % NB == k))
            def _(k=k, n4=n4):
                meta_issue(n4, bufs[k])

        n2 = c + 3
        for k in range(NB):
            @pl.when((n2 < NCH) & (n2 % NB == k))
            def _(k=k, n2=n2):
                gather_issue(n2, bufs[k])

        for k in range(NB):
            @pl.when(c % NB == k)
            def _(k=k):
                consume(bufs[k])
        return 0
    lax.fori_loop(0, NCH, chunk_body, 0)

    pltpu.sync_copy(acc8, out_hbm.at[cid, sid])


def _sc_reduce(y, gi, s):
    ring = [
        pltpu.VMEM((CHUNK,), jnp.int32),
        pltpu.VMEM((CHUNK,), jnp.int32),
        pltpu.VMEM((CHUNK,), jnp.float32),
        pltpu.VMEM((CHUNK, D_HID), jnp.float32),
    ] * 6
    sems = [pltpu.SemaphoreType.DMA] * 12
    return pl.kernel(
        _reduce_body,
        out_type=jax.ShapeDtypeStruct((NC, NS, 8, D_HID), jnp.float32),
        mesh=_mesh,
        compiler_params=_sc_params,
        scratch_types=ring + sems + [pltpu.VMEM((8, D_HID), jnp.float32)],
    )(y, gi, s)


def _tc1_body(x_ref, w_ref, root_ref, b_ref, y_ref, h0_ref):
    xb = x_ref[...]
    for r in range(N_REL):
        y_ref[:, r, :] = jnp.dot(xb, w_ref[r], preferred_element_type=jnp.float32)
    h0_ref[...] = (jnp.dot(xb, root_ref[...], preferred_element_type=jnp.float32)
                   + b_ref[...])


def _tc_transform1(x, W, root, b, d_in, d_out):
    blk = 400
    grid = (N_NODES // blk,)
    return pl.pallas_call(
        _tc1_body,
        grid=grid,
        in_specs=[
            pl.BlockSpec((blk, d_in), lambda i: (i, 0)),
            pl.BlockSpec((N_REL, d_in, d_out), lambda i: (0, 0, 0)),
            pl.BlockSpec((d_in, d_out), lambda i: (0, 0)),
            pl.BlockSpec((1, d_out), lambda i: (0, 0)),
        ],
        out_specs=[
            pl.BlockSpec((blk, N_REL, d_out), lambda i: (i, 0, 0)),
            pl.BlockSpec((blk, d_out), lambda i: (i, 0)),
        ],
        out_shape=[
            jax.ShapeDtypeStruct((N_NODES, N_REL, d_out), jnp.float32),
            jax.ShapeDtypeStruct((N_NODES, d_out), jnp.float32),
        ],
        compiler_params=pltpu.CompilerParams(
            dimension_semantics=("parallel",)),
    )(x, W, root, b)


def _tc2_body(h0_ref, pa_ref, pb_ref, w_ref, y_ref, hs_ref):
    h = jnp.maximum(h0_ref[...] + pa_ref[...] + pb_ref[...], 0.0)
    for rp in range(N_REL // 2):
        lo = jnp.dot(h, w_ref[2 * rp], preferred_element_type=jnp.float32)
        hi = jnp.dot(h, w_ref[2 * rp + 1], preferred_element_type=jnp.float32)
        y_ref[:, rp, :] = jnp.concatenate([lo, hi], axis=-1)
    hs_ref[...] = jnp.sum(h, axis=0, keepdims=True)[None]


def _tc_transform2(h0, pa, pb, W, d_in):
    blk = 400
    grid = (N_NODES // blk,)
    return pl.pallas_call(
        _tc2_body,
        grid=grid,
        in_specs=[
            pl.BlockSpec((blk, d_in), lambda i: (i, 0)),
            pl.BlockSpec((blk, d_in), lambda i: (i, 0)),
            pl.BlockSpec((blk, d_in), lambda i: (i, 0)),
            pl.BlockSpec((N_REL, d_in, D_OUT), lambda i: (0, 0, 0)),
        ],
        out_specs=[
            pl.BlockSpec((blk, N_REL // 2, 2 * D_OUT), lambda i: (i, 0, 0)),
            pl.BlockSpec((1, 1, d_in), lambda i: (i, 0, 0)),
        ],
        out_shape=[
            jax.ShapeDtypeStruct((N_NODES, N_REL // 2, 2 * D_OUT), jnp.float32),
            jax.ShapeDtypeStruct((N_NODES // blk, 1, d_in), jnp.float32),
        ],
        compiler_params=pltpu.CompilerParams(
            dimension_semantics=("parallel",)),
    )(h0, pa, pb, W)


def _fin_body(hs_ref, root_ref, b_ref, ep_ref, o_ref):
    hv = jnp.sum(hs_ref[...], axis=(0, 1))[None, :]
    ep = jnp.sum(ep_ref[...], axis=(0, 1, 2))[None, :D_OUT]
    g = (jnp.dot(hv, root_ref[...], preferred_element_type=jnp.float32)
         + ep) / N_NODES + b_ref[...]
    m = jnp.max(g)
    lse = jnp.log(jnp.sum(jnp.exp(g - m))) + m
    o_ref[...] = g - lse


def _tc_finish(hsum, root, b, ep):
    nb = hsum.shape[0]
    return pl.pallas_call(
        _fin_body,
        in_specs=[
            pl.BlockSpec((nb, 1, D_HID), lambda: (0, 0, 0)),
            pl.BlockSpec((D_HID, D_OUT), lambda: (0, 0)),
            pl.BlockSpec((1, D_OUT), lambda: (0, 0)),
            pl.BlockSpec((NC, NS, 8, D_HID), lambda: (0, 0, 0, 0)),
        ],
        out_specs=pl.BlockSpec((1, D_OUT), lambda: (0, 0)),
        out_shape=jax.ShapeDtypeStruct((1, D_OUT), jnp.float32),
    )(hsum, root, b, ep)


def kernel(x, edge_index, edge_type, W1, root1, b1, W2, root2, b2):
    src = edge_index[0]
    dst = edge_index[1]
    et = edge_type

    s, gi = _sc_scales(et, dst, src)

    y1, h0 = _tc_transform1(x, W1, root1, b1.reshape(1, D_HID), D_IN, D_HID)
    p1 = _sc_scatter(y1.reshape(N_NODES * N_REL, D_HID), gi, dst, s, D_HID)

    y2, hsum = _tc_transform2(h0, p1[0], p1[1], W2, D_HID)
    ep = _sc_reduce(y2.reshape(N_NODES * N_REL // 2, 2 * D_OUT), gi, s)

    return _tc_finish(hsum, root2, b2.reshape(1, D_OUT), ep)
